# Initial kernel scaffold; baseline (speedup 1.0000x reference)
import jax
import jax.numpy as jnp
from jax.experimental import pallas as pl


def kernel(x, edge_index, batch, W1, b1, ln1_w, ln1_b, W2, b2, ln2_w, ln2_b, Wh, bh):
    raise NotImplementedError("write your pallas kernel here")


# speedup vs baseline: 21.0824x; 21.0824x over previous
import functools

import jax
import jax.numpy as jnp
from jax import lax
from jax.experimental import pallas as pl
from jax.experimental.pallas import tpu as pltpu
from jax.experimental.pallas import tpu_sc as plsc

N = 10000
E = 640000
IN_DIM = 128
HIDDEN = 64
EPS = 1e-5

NC = 2
NS = 16
NW = NC * NS
EPW = E // NW
CH = 128
NFULL = EPW // CH
REM = EPW - NFULL * CH
RPT = 632
RPT_LAST = N - (NS - 1) * RPT

_mesh = plsc.VectorSubcoreMesh(core_axis_name="c", subcore_axis_name="s")


@functools.partial(
    pl.kernel,
    out_type=jax.ShapeDtypeStruct((NC, N), jnp.float32),
    mesh=_mesh,
    scratch_types=[
        pltpu.VMEM((CH,), jnp.int32),
        pltpu.VMEM((REM,), jnp.int32),
        pltpu.VMEM((CH,), jnp.float32),
        pltpu.VMEM_SHARED((N,), jnp.float32),
    ],
)
def _deg_kernel(dst_hbm, zeros_hbm, out_hbm, idx_v, idxr_v, ones_v, deg_sh):
    cid = lax.axis_index("c")
    sid = lax.axis_index("s")
    wid = cid * NS + sid

    for i in range(CH // 16):
        ones_v[pl.ds(i * 16, 16)] = jnp.ones((16,), jnp.float32)

    @pl.when(sid == 0)
    def _():
        pltpu.sync_copy(zeros_hbm, deg_sh)

    plsc.subcore_barrier()

    def body(c, _):
        base = pl.multiple_of(wid * EPW + c * CH, 8)
        pltpu.sync_copy(dst_hbm.at[pl.ds(base, CH)], idx_v)
        pltpu.sync_copy(ones_v, deg_sh.at[idx_v], add=True)
        return 0

    lax.fori_loop(0, NFULL, body, 0)

    base = pl.multiple_of(wid * EPW + NFULL * CH, 8)
    pltpu.sync_copy(dst_hbm.at[pl.ds(base, REM)], idxr_v)
    pltpu.sync_copy(ones_v.at[pl.ds(0, REM)], deg_sh.at[idxr_v], add=True)

    plsc.subcore_barrier()

    @pl.when(sid == 0)
    def _():
        pltpu.sync_copy(deg_sh, out_hbm.at[cid])


@functools.partial(
    pl.kernel,
    out_type=jax.ShapeDtypeStruct((NC, N, HIDDEN), jnp.float32),
    mesh=_mesh,
    scratch_types=[
        pltpu.VMEM((CH,), jnp.int32),
        pltpu.VMEM((CH,), jnp.int32),
        pltpu.VMEM((REM,), jnp.int32),
        pltpu.VMEM((REM,), jnp.int32),
        pltpu.VMEM((CH, HIDDEN), jnp.float32),
        pltpu.VMEM((REM, HIDDEN), jnp.float32),
        pltpu.VMEM_SHARED((N, HIDDEN), jnp.float32),
        pltpu.SemaphoreType.DMA,
    ],
    compiler_params=pltpu.CompilerParams(use_tc_tiling_on_sc=False),
)
def _msg_kernel(g_hbm, src_hbm, dst_hbm, zeros_hbm, out_hbm,
                src_v, dst_v, srcr_v, dstr_v, msg_v, msgr_v, acc_sh, sem):
    cid = lax.axis_index("c")
    sid = lax.axis_index("s")
    wid = cid * NS + sid

    @pl.when(sid < NS - 1)
    def _():
        pltpu.sync_copy(zeros_hbm.at[pl.ds(sid * RPT, RPT)],
                        acc_sh.at[pl.ds(sid * RPT, RPT)])

    @pl.when(sid == NS - 1)
    def _():
        pltpu.sync_copy(zeros_hbm.at[pl.ds((NS - 1) * RPT, RPT_LAST)],
                        acc_sh.at[pl.ds((NS - 1) * RPT, RPT_LAST)])

    plsc.subcore_barrier()

    def body(c, _):
        base = pl.multiple_of(wid * EPW + c * CH, 8)
        pltpu.sync_copy(src_hbm.at[pl.ds(base, CH)], src_v)
        pltpu.sync_copy(dst_hbm.at[pl.ds(base, CH)], dst_v)
        pltpu.async_copy(g_hbm.at[src_v], msg_v, sem).wait()
        pltpu.sync_copy(msg_v, acc_sh.at[dst_v], add=True)
        return 0

    lax.fori_loop(0, NFULL, body, 0)

    base = pl.multiple_of(wid * EPW + NFULL * CH, 8)
    pltpu.sync_copy(src_hbm.at[pl.ds(base, REM)], srcr_v)
    pltpu.sync_copy(dst_hbm.at[pl.ds(base, REM)], dstr_v)
    pltpu.async_copy(g_hbm.at[srcr_v], msgr_v, sem).wait()
    pltpu.sync_copy(msgr_v, acc_sh.at[dstr_v], add=True)

    plsc.subcore_barrier()

    @pl.when(sid < NS - 1)
    def _():
        pltpu.sync_copy(acc_sh.at[pl.ds(sid * RPT, RPT)],
                        out_hbm.at[cid, pl.ds(sid * RPT, RPT)])

    @pl.when(sid == NS - 1)
    def _():
        pltpu.sync_copy(acc_sh.at[pl.ds((NS - 1) * RPT, RPT_LAST)],
                        out_hbm.at[cid, pl.ds((NS - 1) * RPT, RPT_LAST)])


def _dinv_from(degt_ref):
    deg = degt_ref[:, 0:1] + degt_ref[:, 1:2] + 1.0
    return lax.rsqrt(deg)


def _tc_pre_body(x_ref, w_ref, degt_ref, g_ref):
    dinv = _dinv_from(degt_ref)
    h = jnp.dot(x_ref[...], w_ref[...], preferred_element_type=jnp.float32)
    g_ref[...] = h * dinv


_tc_pre = pl.pallas_call(
    _tc_pre_body,
    out_shape=jax.ShapeDtypeStruct((N, HIDDEN), jnp.float32),
)


def _ln_relu(u, lw, lb):
    m = jnp.mean(u)
    xc = u - m
    v = jnp.mean(xc * xc)
    yn = xc / (jnp.sqrt(v) + EPS) * lw + lb
    return jnp.maximum(yn, 0.0)


def _tc_mid_body(s_ref, g_ref, degt_ref, b_ref, lw_ref, lb_ref, w2_ref, out_ref):
    dinv = _dinv_from(degt_ref)
    u = (s_ref[0] + s_ref[1] + g_ref[...]) * dinv + b_ref[...]
    yr = _ln_relu(u, lw_ref[...], lb_ref[...])
    h2 = jnp.dot(yr, w2_ref[...], preferred_element_type=jnp.float32)
    out_ref[...] = h2 * dinv


_tc_mid = pl.pallas_call(
    _tc_mid_body,
    out_shape=jax.ShapeDtypeStruct((N, HIDDEN), jnp.float32),
)


def _tc_fin_body(s_ref, g_ref, degt_ref, b_ref, lw_ref, lb_ref, wh_ref, bh_ref,
                 out_ref):
    dinv = _dinv_from(degt_ref)
    u = (s_ref[0] + s_ref[1] + g_ref[...]) * dinv + b_ref[...]
    yr = _ln_relu(u, lw_ref[...], lb_ref[...])
    out_ref[...] = jnp.dot(yr, wh_ref[...], preferred_element_type=jnp.float32) + bh_ref[...]


_tc_fin = pl.pallas_call(
    _tc_fin_body,
    out_shape=jax.ShapeDtypeStruct((N, 1), jnp.float32),
)


def kernel(x, edge_index, batch, W1, b1, ln1_w, ln1_b, W2, b2, ln2_w, ln2_b,
           Wh, bh):
    src = edge_index[0]
    dst = edge_index[1]
    zeros1 = jnp.zeros((N,), jnp.float32)
    zeros64 = jnp.zeros((N, HIDDEN), jnp.float32)

    degp = _deg_kernel(dst, zeros1)
    degt = degp.T

    b1r = b1.reshape(1, HIDDEN)
    lw1r = ln1_w.reshape(1, HIDDEN)
    lb1r = ln1_b.reshape(1, HIDDEN)
    b2r = b2.reshape(1, HIDDEN)
    lw2r = ln2_w.reshape(1, HIDDEN)
    lb2r = ln2_b.reshape(1, HIDDEN)
    bhr = bh.reshape(1, 1)

    g1 = _tc_pre(x, W1, degt)
    s1 = _msg_kernel(g1, src, dst, zeros64)
    g2 = _tc_mid(s1, g1, degt, b1r, lw1r, lb1r, W2)
    s2 = _msg_kernel(g2, src, dst, zeros64)
    return _tc_fin(s2, g2, degt, b2r, lw2r, lb2r, Wh, bhr)


# speedup vs baseline: 37.8697x; 1.7963x over previous
import functools

import jax
import jax.numpy as jnp
from jax import lax
from jax.experimental import pallas as pl
from jax.experimental.pallas import tpu as pltpu
from jax.experimental.pallas import tpu_sc as plsc

N = 10000
E = 640000
IN_DIM = 128
HIDDEN = 64
EPS = 1e-5

NC = 2
NS = 16
NW = NC * NS
EPW = E // NW
CH = 128
NFULL = EPW // CH
REM = EPW - NFULL * CH
RPT = 632
RPT_LAST = N - (NS - 1) * RPT

_mesh = plsc.VectorSubcoreMesh(core_axis_name="c", subcore_axis_name="s")


@functools.partial(
    pl.kernel,
    out_type=jax.ShapeDtypeStruct((NC, N), jnp.float32),
    mesh=_mesh,
    scratch_types=[
        pltpu.VMEM((CH,), jnp.int32),
        pltpu.VMEM((CH,), jnp.int32),
        pltpu.VMEM((REM,), jnp.int32),
        pltpu.VMEM((CH,), jnp.float32),
        pltpu.VMEM_SHARED((N,), jnp.float32),
        pltpu.SemaphoreType.DMA,
        pltpu.SemaphoreType.DMA,
    ],
)
def _deg_kernel(dst_hbm, zeros_hbm, out_hbm, idx_v0, idx_v1, idxr_v, ones_v,
                deg_sh, sem_0, sem_1):
    cid = lax.axis_index("c")
    sid = lax.axis_index("s")
    wid = cid * NS + sid

    idx_v = (idx_v0, idx_v1)
    sem = (sem_0, sem_1)

    def start_idx(c, k):
        base = pl.multiple_of(wid * EPW + c * CH, 8)
        pltpu.async_copy(dst_hbm.at[pl.ds(base, CH)], idx_v[k], sem[k])

    def wait_idx(k):
        pltpu.make_async_copy(dst_hbm.at[pl.ds(0, CH)], idx_v[k], sem[k]).wait()

    for i in range(CH // 16):
        ones_v[pl.ds(i * 16, 16)] = jnp.ones((16,), jnp.float32)

    @pl.when(sid == 0)
    def _():
        pltpu.sync_copy(zeros_hbm, deg_sh)

    plsc.subcore_barrier()

    start_idx(0, 0)

    def outer(m, _):
        for b in (0, 1):
            c = 2 * m + b
            nb = 1 - b
            wait_idx(b)

            @pl.when(c < NFULL - 1)
            def _():
                start_idx(c + 1, nb)

            pltpu.sync_copy(ones_v, deg_sh.at[idx_v[b]], add=True)
        return 0

    lax.fori_loop(0, NFULL // 2, outer, 0)

    base = pl.multiple_of(wid * EPW + NFULL * CH, 8)
    pltpu.sync_copy(dst_hbm.at[pl.ds(base, REM)], idxr_v)
    pltpu.sync_copy(ones_v.at[pl.ds(0, REM)], deg_sh.at[idxr_v], add=True)

    plsc.subcore_barrier()

    @pl.when(sid == 0)
    def _():
        pltpu.sync_copy(deg_sh, out_hbm.at[cid])


@functools.partial(
    pl.kernel,
    out_type=jax.ShapeDtypeStruct((NC, N, HIDDEN), jnp.float32),
    mesh=_mesh,
    scratch_types=[
        pltpu.VMEM((CH,), jnp.int32),
        pltpu.VMEM((CH,), jnp.int32),
        pltpu.VMEM((CH,), jnp.int32),
        pltpu.VMEM((CH,), jnp.int32),
        pltpu.VMEM((REM,), jnp.int32),
        pltpu.VMEM((REM,), jnp.int32),
        pltpu.VMEM((CH, HIDDEN), jnp.float32),
        pltpu.VMEM((CH, HIDDEN), jnp.float32),
        pltpu.VMEM((REM, HIDDEN), jnp.float32),
        pltpu.VMEM_SHARED((N, HIDDEN), jnp.float32),
        pltpu.SemaphoreType.DMA,
        pltpu.SemaphoreType.DMA,
        pltpu.SemaphoreType.DMA,
        pltpu.SemaphoreType.DMA,
        pltpu.SemaphoreType.DMA,
        pltpu.SemaphoreType.DMA,
    ],
    compiler_params=pltpu.CompilerParams(use_tc_tiling_on_sc=False),
)
def _msg_kernel(g_hbm, src_hbm, dst_hbm, zeros_hbm, out_hbm,
                src_v0, src_v1, dst_v0, dst_v1, srcr_v, dstr_v,
                msg_v0, msg_v1, msgr_v, acc_sh,
                sem_s0, sem_s1, sem_d0, sem_d1, sem_g0, sem_g1):
    cid = lax.axis_index("c")
    sid = lax.axis_index("s")
    wid = cid * NS + sid

    src_v = (src_v0, src_v1)
    dst_v = (dst_v0, dst_v1)
    msg_v = (msg_v0, msg_v1)
    sem_s = (sem_s0, sem_s1)
    sem_d = (sem_d0, sem_d1)
    sem_g = (sem_g0, sem_g1)

    def start_idx(c, k):
        base = pl.multiple_of(wid * EPW + c * CH, 8)
        pltpu.async_copy(src_hbm.at[pl.ds(base, CH)], src_v[k], sem_s[k])
        pltpu.async_copy(dst_hbm.at[pl.ds(base, CH)], dst_v[k], sem_d[k])

    def wait_idx(k):
        pltpu.make_async_copy(src_hbm.at[pl.ds(0, CH)], src_v[k], sem_s[k]).wait()
        pltpu.make_async_copy(dst_hbm.at[pl.ds(0, CH)], dst_v[k], sem_d[k]).wait()

    def start_gather(k):
        pltpu.async_copy(g_hbm.at[src_v[k]], msg_v[k], sem_g[k])

    def wait_gather(k):
        pltpu.make_async_copy(g_hbm.at[src_v[k]], msg_v[k], sem_g[k]).wait()

    @pl.when(sid < NS - 1)
    def _():
        pltpu.sync_copy(zeros_hbm.at[pl.ds(sid * RPT, RPT)],
                        acc_sh.at[pl.ds(sid * RPT, RPT)])

    @pl.when(sid == NS - 1)
    def _():
        pltpu.sync_copy(zeros_hbm.at[pl.ds((NS - 1) * RPT, RPT_LAST)],
                        acc_sh.at[pl.ds((NS - 1) * RPT, RPT_LAST)])

    plsc.subcore_barrier()

    start_idx(0, 0)
    wait_idx(0)
    start_gather(0)
    start_idx(1, 1)

    def outer(m, _):
        for b in (0, 1):
            c = 2 * m + b
            nb = 1 - b

            @pl.when(c < NFULL - 1)
            def _():
                wait_idx(nb)

            wait_gather(b)

            @pl.when(c < NFULL - 1)
            def _():
                start_gather(nb)

            pltpu.sync_copy(msg_v[b], acc_sh.at[dst_v[b]], add=True)

            @pl.when(c < NFULL - 2)
            def _():
                start_idx(c + 2, b)
        return 0

    lax.fori_loop(0, NFULL // 2, outer, 0)

    base = pl.multiple_of(wid * EPW + NFULL * CH, 8)
    pltpu.sync_copy(src_hbm.at[pl.ds(base, REM)], srcr_v)
    pltpu.sync_copy(dst_hbm.at[pl.ds(base, REM)], dstr_v)
    pltpu.async_copy(g_hbm.at[srcr_v], msgr_v, sem_g0).wait()
    pltpu.sync_copy(msgr_v, acc_sh.at[dstr_v], add=True)

    plsc.subcore_barrier()

    @pl.when(sid < NS - 1)
    def _():
        pltpu.sync_copy(acc_sh.at[pl.ds(sid * RPT, RPT)],
                        out_hbm.at[cid, pl.ds(sid * RPT, RPT)])

    @pl.when(sid == NS - 1)
    def _():
        pltpu.sync_copy(acc_sh.at[pl.ds((NS - 1) * RPT, RPT_LAST)],
                        out_hbm.at[cid, pl.ds((NS - 1) * RPT, RPT_LAST)])


def _dinv_from(degt_ref):
    deg = degt_ref[:, 0:1] + degt_ref[:, 1:2] + 1.0
    return lax.rsqrt(deg)


def _tc_pre_body(x_ref, w_ref, degt_ref, g_ref):
    dinv = _dinv_from(degt_ref)
    h = jnp.dot(x_ref[...], w_ref[...], preferred_element_type=jnp.float32)
    g_ref[...] = h * dinv


_tc_pre = pl.pallas_call(
    _tc_pre_body,
    out_shape=jax.ShapeDtypeStruct((N, HIDDEN), jnp.float32),
)


def _ln_relu(u, lw, lb):
    m = jnp.mean(u)
    xc = u - m
    v = jnp.mean(xc * xc)
    yn = xc / (jnp.sqrt(v) + EPS) * lw + lb
    return jnp.maximum(yn, 0.0)


def _tc_mid_body(s_ref, g_ref, degt_ref, b_ref, lw_ref, lb_ref, w2_ref, out_ref):
    dinv = _dinv_from(degt_ref)
    u = (s_ref[0] + s_ref[1] + g_ref[...]) * dinv + b_ref[...]
    yr = _ln_relu(u, lw_ref[...], lb_ref[...])
    h2 = jnp.dot(yr, w2_ref[...], preferred_element_type=jnp.float32)
    out_ref[...] = h2 * dinv


_tc_mid = pl.pallas_call(
    _tc_mid_body,
    out_shape=jax.ShapeDtypeStruct((N, HIDDEN), jnp.float32),
)


def _tc_fin_body(s_ref, g_ref, degt_ref, b_ref, lw_ref, lb_ref, wh_ref, bh_ref,
                 out_ref):
    dinv = _dinv_from(degt_ref)
    u = (s_ref[0] + s_ref[1] + g_ref[...]) * dinv + b_ref[...]
    yr = _ln_relu(u, lw_ref[...], lb_ref[...])
    out_ref[...] = jnp.dot(yr, wh_ref[...], preferred_element_type=jnp.float32) + bh_ref[...]


_tc_fin = pl.pallas_call(
    _tc_fin_body,
    out_shape=jax.ShapeDtypeStruct((N, 1), jnp.float32),
)


def kernel(x, edge_index, batch, W1, b1, ln1_w, ln1_b, W2, b2, ln2_w, ln2_b,
           Wh, bh):
    src = edge_index[0]
    dst = edge_index[1]
    zeros1 = jnp.zeros((N,), jnp.float32)
    zeros64 = jnp.zeros((N, HIDDEN), jnp.float32)

    degp = _deg_kernel(dst, zeros1)
    degt = degp.T

    b1r = b1.reshape(1, HIDDEN)
    lw1r = ln1_w.reshape(1, HIDDEN)
    lb1r = ln1_b.reshape(1, HIDDEN)
    b2r = b2.reshape(1, HIDDEN)
    lw2r = ln2_w.reshape(1, HIDDEN)
    lb2r = ln2_b.reshape(1, HIDDEN)
    bhr = bh.reshape(1, 1)

    g1 = _tc_pre(x, W1, degt)
    s1 = _msg_kernel(g1, src, dst, zeros64)
    g2 = _tc_mid(s1, g1, degt, b1r, lw1r, lb1r, W2)
    s2 = _msg_kernel(g2, src, dst, zeros64)
    return _tc_fin(s2, g2, degt, b2r, lw2r, lb2r, Wh, bhr)


# speedup vs baseline: 37.9067x; 1.0010x over previous
import functools

import jax
import jax.numpy as jnp
from jax import lax
from jax.experimental import pallas as pl
from jax.experimental.pallas import tpu as pltpu
from jax.experimental.pallas import tpu_sc as plsc

N = 10000
E = 640000
IN_DIM = 128
HIDDEN = 64
EPS = 1e-5

NC = 2
NS = 16
NW = NC * NS
EPW = E // NW
CH = 128
NFULL = EPW // CH
REM = EPW - NFULL * CH
RPT = 632
RPT_LAST = N - (NS - 1) * RPT

_mesh = plsc.VectorSubcoreMesh(core_axis_name="c", subcore_axis_name="s")


@functools.partial(
    pl.kernel,
    out_type=jax.ShapeDtypeStruct((NC, N), jnp.float32),
    mesh=_mesh,
    scratch_types=[
        pltpu.VMEM((CH,), jnp.int32),
        pltpu.VMEM((CH,), jnp.int32),
        pltpu.VMEM((REM,), jnp.int32),
        pltpu.VMEM((CH,), jnp.float32),
        pltpu.VMEM_SHARED((N,), jnp.float32),
        pltpu.SemaphoreType.DMA,
        pltpu.SemaphoreType.DMA,
    ],
)
def _deg_kernel(dst_hbm, zeros_hbm, out_hbm, idx_v0, idx_v1, idxr_v, ones_v,
                deg_sh, sem_0, sem_1):
    cid = lax.axis_index("c")
    sid = lax.axis_index("s")
    wid = cid * NS + sid

    idx_v = (idx_v0, idx_v1)
    sem = (sem_0, sem_1)

    def start_idx(c, k):
        base = pl.multiple_of(wid * EPW + c * CH, 8)
        pltpu.async_copy(dst_hbm.at[pl.ds(base, CH)], idx_v[k], sem[k])

    def wait_idx(k):
        pltpu.make_async_copy(dst_hbm.at[pl.ds(0, CH)], idx_v[k], sem[k]).wait()

    for i in range(CH // 16):
        ones_v[pl.ds(i * 16, 16)] = jnp.ones((16,), jnp.float32)

    @pl.when(sid == 0)
    def _():
        pltpu.sync_copy(zeros_hbm, deg_sh)

    plsc.subcore_barrier()

    start_idx(0, 0)

    def outer(m, _):
        for b in (0, 1):
            c = 2 * m + b
            nb = 1 - b
            wait_idx(b)

            @pl.when(c < NFULL - 1)
            def _():
                start_idx(c + 1, nb)

            pltpu.sync_copy(ones_v, deg_sh.at[idx_v[b]], add=True)
        return 0

    lax.fori_loop(0, NFULL // 2, outer, 0)

    base = pl.multiple_of(wid * EPW + NFULL * CH, 8)
    pltpu.sync_copy(dst_hbm.at[pl.ds(base, REM)], idxr_v)
    pltpu.sync_copy(ones_v.at[pl.ds(0, REM)], deg_sh.at[idxr_v], add=True)

    plsc.subcore_barrier()

    @pl.when(sid == 0)
    def _():
        pltpu.sync_copy(deg_sh, out_hbm.at[cid])


@functools.partial(
    pl.kernel,
    out_type=jax.ShapeDtypeStruct((NC, N, HIDDEN), jnp.float32),
    mesh=_mesh,
    scratch_types=[
        pltpu.VMEM((CH,), jnp.int32),
        pltpu.VMEM((CH,), jnp.int32),
        pltpu.VMEM((CH,), jnp.int32),
        pltpu.VMEM((CH,), jnp.int32),
        pltpu.VMEM((CH,), jnp.int32),
        pltpu.VMEM((CH,), jnp.int32),
        pltpu.VMEM((REM,), jnp.int32),
        pltpu.VMEM((REM,), jnp.int32),
        pltpu.VMEM((CH, HIDDEN), jnp.float32),
        pltpu.VMEM((CH, HIDDEN), jnp.float32),
        pltpu.VMEM((CH, HIDDEN), jnp.float32),
        pltpu.VMEM((REM, HIDDEN), jnp.float32),
        pltpu.VMEM_SHARED((N, HIDDEN), jnp.float32),
        pltpu.SemaphoreType.DMA,
        pltpu.SemaphoreType.DMA,
        pltpu.SemaphoreType.DMA,
        pltpu.SemaphoreType.DMA,
        pltpu.SemaphoreType.DMA,
        pltpu.SemaphoreType.DMA,
        pltpu.SemaphoreType.DMA,
        pltpu.SemaphoreType.DMA,
        pltpu.SemaphoreType.DMA,
        pltpu.SemaphoreType.DMA,
        pltpu.SemaphoreType.DMA,
        pltpu.SemaphoreType.DMA,
    ],
    compiler_params=pltpu.CompilerParams(use_tc_tiling_on_sc=False),
)
def _msg_kernel(g_hbm, src_hbm, dst_hbm, zeros_hbm, out_hbm,
                src_v0, src_v1, src_v2, dst_v0, dst_v1, dst_v2, srcr_v, dstr_v,
                msg_v0, msg_v1, msg_v2, msgr_v, acc_sh,
                sem_s0, sem_s1, sem_s2, sem_d0, sem_d1, sem_d2,
                sem_g0, sem_g1, sem_g2, sem_a0, sem_a1, sem_a2):
    cid = lax.axis_index("c")
    sid = lax.axis_index("s")
    wid = cid * NS + sid

    src_v = (src_v0, src_v1, src_v2)
    dst_v = (dst_v0, dst_v1, dst_v2)
    msg_v = (msg_v0, msg_v1, msg_v2)
    sem_s = (sem_s0, sem_s1, sem_s2)
    sem_d = (sem_d0, sem_d1, sem_d2)
    sem_g = (sem_g0, sem_g1, sem_g2)
    sem_a = (sem_a0, sem_a1, sem_a2)

    def start_idx(c, k):
        base = pl.multiple_of(wid * EPW + c * CH, 8)
        pltpu.async_copy(src_hbm.at[pl.ds(base, CH)], src_v[k], sem_s[k])
        pltpu.async_copy(dst_hbm.at[pl.ds(base, CH)], dst_v[k], sem_d[k])

    def wait_idx(k):
        pltpu.make_async_copy(src_hbm.at[pl.ds(0, CH)], src_v[k], sem_s[k]).wait()
        pltpu.make_async_copy(dst_hbm.at[pl.ds(0, CH)], dst_v[k], sem_d[k]).wait()

    def start_gather(k):
        pltpu.async_copy(g_hbm.at[src_v[k]], msg_v[k], sem_g[k])

    def wait_gather(k):
        pltpu.make_async_copy(g_hbm.at[src_v[k]], msg_v[k], sem_g[k]).wait()

    def start_scatter(k):
        pltpu.async_copy(msg_v[k], acc_sh.at[dst_v[k]], sem_a[k], add=True)

    def wait_scatter(k):
        pltpu.make_async_copy(msg_v[k], acc_sh.at[dst_v[k]], sem_a[k]).wait()

    @pl.when(sid < NS - 1)
    def _():
        pltpu.sync_copy(zeros_hbm.at[pl.ds(sid * RPT, RPT)],
                        acc_sh.at[pl.ds(sid * RPT, RPT)])

    @pl.when(sid == NS - 1)
    def _():
        pltpu.sync_copy(zeros_hbm.at[pl.ds((NS - 1) * RPT, RPT_LAST)],
                        acc_sh.at[pl.ds((NS - 1) * RPT, RPT_LAST)])

    plsc.subcore_barrier()

    start_idx(0, 0)
    start_idx(1, 1)
    wait_idx(0)
    start_gather(0)

    def outer(m, _):
        for b in (0, 1, 2):
            c = 3 * m + b
            b1 = (b + 1) % 3
            b2 = (b + 2) % 3

            wait_gather(b)
            start_scatter(b)

            @pl.when(c < NFULL - 1)
            def _():
                wait_idx(b1)
                start_gather(b1)

            @pl.when(c >= 1)
            def _():
                wait_scatter(b2)

            @pl.when(c < NFULL - 2)
            def _():
                start_idx(c + 2, b2)
        return 0

    lax.fori_loop(0, NFULL // 3, outer, 0)

    wait_scatter((NFULL - 1) % 3)

    base = pl.multiple_of(wid * EPW + NFULL * CH, 8)
    pltpu.sync_copy(src_hbm.at[pl.ds(base, REM)], srcr_v)
    pltpu.sync_copy(dst_hbm.at[pl.ds(base, REM)], dstr_v)
    pltpu.async_copy(g_hbm.at[srcr_v], msgr_v, sem_g0).wait()
    pltpu.sync_copy(msgr_v, acc_sh.at[dstr_v], add=True)

    plsc.subcore_barrier()

    @pl.when(sid < NS - 1)
    def _():
        pltpu.sync_copy(acc_sh.at[pl.ds(sid * RPT, RPT)],
                        out_hbm.at[cid, pl.ds(sid * RPT, RPT)])

    @pl.when(sid == NS - 1)
    def _():
        pltpu.sync_copy(acc_sh.at[pl.ds((NS - 1) * RPT, RPT_LAST)],
                        out_hbm.at[cid, pl.ds((NS - 1) * RPT, RPT_LAST)])


def _dinv_from(degt_ref):
    deg = degt_ref[:, 0:1] + degt_ref[:, 1:2] + 1.0
    return lax.rsqrt(deg)


def _tc_pre_body(x_ref, w_ref, degt_ref, g_ref):
    dinv = _dinv_from(degt_ref)
    h = jnp.dot(x_ref[...], w_ref[...], preferred_element_type=jnp.float32)
    g_ref[...] = h * dinv


_tc_pre = pl.pallas_call(
    _tc_pre_body,
    out_shape=jax.ShapeDtypeStruct((N, HIDDEN), jnp.float32),
)


def _ln_relu(u, lw, lb):
    m = jnp.mean(u)
    xc = u - m
    v = jnp.mean(xc * xc)
    yn = xc / (jnp.sqrt(v) + EPS) * lw + lb
    return jnp.maximum(yn, 0.0)


def _tc_mid_body(s_ref, g_ref, degt_ref, b_ref, lw_ref, lb_ref, w2_ref, out_ref):
    dinv = _dinv_from(degt_ref)
    u = (s_ref[0] + s_ref[1] + g_ref[...]) * dinv + b_ref[...]
    yr = _ln_relu(u, lw_ref[...], lb_ref[...])
    h2 = jnp.dot(yr, w2_ref[...], preferred_element_type=jnp.float32)
    out_ref[...] = h2 * dinv


_tc_mid = pl.pallas_call(
    _tc_mid_body,
    out_shape=jax.ShapeDtypeStruct((N, HIDDEN), jnp.float32),
)


def _tc_fin_body(s_ref, g_ref, degt_ref, b_ref, lw_ref, lb_ref, wh_ref, bh_ref,
                 out_ref):
    dinv = _dinv_from(degt_ref)
    u = (s_ref[0] + s_ref[1] + g_ref[...]) * dinv + b_ref[...]
    yr = _ln_relu(u, lw_ref[...], lb_ref[...])
    out_ref[...] = jnp.dot(yr, wh_ref[...], preferred_element_type=jnp.float32) + bh_ref[...]


_tc_fin = pl.pallas_call(
    _tc_fin_body,
    out_shape=jax.ShapeDtypeStruct((N, 1), jnp.float32),
)


def kernel(x, edge_index, batch, W1, b1, ln1_w, ln1_b, W2, b2, ln2_w, ln2_b,
           Wh, bh):
    src = edge_index[0]
    dst = edge_index[1]
    zeros1 = jnp.zeros((N,), jnp.float32)
    zeros64 = jnp.zeros((N, HIDDEN), jnp.float32)

    degp = _deg_kernel(dst, zeros1)
    degt = degp.T

    b1r = b1.reshape(1, HIDDEN)
    lw1r = ln1_w.reshape(1, HIDDEN)
    lb1r = ln1_b.reshape(1, HIDDEN)
    b2r = b2.reshape(1, HIDDEN)
    lw2r = ln2_w.reshape(1, HIDDEN)
    lb2r = ln2_b.reshape(1, HIDDEN)
    bhr = bh.reshape(1, 1)

    g1 = _tc_pre(x, W1, degt)
    s1 = _msg_kernel(g1, src, dst, zeros64)
    g2 = _tc_mid(s1, g1, degt, b1r, lw1r, lb1r, W2)
    s2 = _msg_kernel(g2, src, dst, zeros64)
    return _tc_fin(s2, g2, degt, b2r, lw2r, lb2r, Wh, bhr)


# speedup vs baseline: 54.8702x; 1.4475x over previous
import functools

import jax
import jax.numpy as jnp
from jax import lax
from jax.experimental import pallas as pl
from jax.experimental.pallas import tpu as pltpu
from jax.experimental.pallas import tpu_sc as plsc

N = 10000
E = 640000
IN_DIM = 128
HIDDEN = 64
EPS = 1e-5

NC = 2
NS = 16
NW = NC * NS
EPW = E // NW
CH = 128
EROWS = E // CH
SLOTS = 78
RPT = 632
RPT_LAST = N - (NS - 1) * RPT

_mesh = plsc.VectorSubcoreMesh(core_axis_name="c", subcore_axis_name="s")


@functools.partial(
    pl.kernel,
    out_type=jax.ShapeDtypeStruct((NW, N), jnp.float32),
    mesh=_mesh,
    scratch_types=[
        pltpu.VMEM((EPW,), jnp.int32),
        pltpu.VMEM((N,), jnp.float32),
        pltpu.SemaphoreType.DMA,
    ],
    compiler_params=pltpu.CompilerParams(needs_layout_passes=False),
)
def _deg_kernel(dst_hbm, out_hbm, idx_v, hist_v, sem):
    cid = lax.axis_index("c")
    sid = lax.axis_index("s")
    wid = cid * NS + sid

    pltpu.async_copy(dst_hbm.at[pl.ds(wid * EPW, EPW)], idx_v, sem)

    zero16 = jnp.zeros((16,), jnp.float32)

    def zbody(i, _):
        hist_v[pl.ds(i * 16, 16)] = zero16
        return 0

    lax.fori_loop(0, N // 16, zbody, 0)

    pltpu.make_async_copy(dst_hbm.at[pl.ds(0, EPW)], idx_v, sem).wait()

    ones16 = jnp.ones((16,), jnp.float32)

    def body(i, _):
        for u in range(5):
            idx16 = idx_v[pl.ds((i * 5 + u) * 16, 16)]
            plsc.addupdate_scatter(hist_v, [idx16], ones16)
        return 0

    lax.fori_loop(0, EPW // (16 * 5), body, 0)

    pltpu.sync_copy(hist_v, out_hbm.at[wid])


@functools.partial(
    pl.kernel,
    out_type=jax.ShapeDtypeStruct((NC, N, HIDDEN), jnp.float32),
    mesh=_mesh,
    scratch_types=[
        pltpu.VMEM((2, CH), jnp.int32),
        pltpu.VMEM((2, CH), jnp.int32),
        pltpu.VMEM((2, CH), jnp.int32),
        pltpu.VMEM((2, CH), jnp.int32),
        pltpu.VMEM((2, CH), jnp.int32),
        pltpu.VMEM((2, CH), jnp.int32),
        pltpu.VMEM((1, CH), jnp.int32),
        pltpu.VMEM((1, CH), jnp.int32),
        pltpu.VMEM((2 * CH, HIDDEN), jnp.float32),
        pltpu.VMEM((2 * CH, HIDDEN), jnp.float32),
        pltpu.VMEM((2 * CH, HIDDEN), jnp.float32),
        pltpu.VMEM((CH, HIDDEN), jnp.float32),
        pltpu.VMEM_SHARED((N, HIDDEN), jnp.float32),
        pltpu.SemaphoreType.DMA,
        pltpu.SemaphoreType.DMA,
        pltpu.SemaphoreType.DMA,
        pltpu.SemaphoreType.DMA,
        pltpu.SemaphoreType.DMA,
        pltpu.SemaphoreType.DMA,
        pltpu.SemaphoreType.DMA,
        pltpu.SemaphoreType.DMA,
        pltpu.SemaphoreType.DMA,
        pltpu.SemaphoreType.DMA,
        pltpu.SemaphoreType.DMA,
        pltpu.SemaphoreType.DMA,
    ],
    compiler_params=pltpu.CompilerParams(use_tc_tiling_on_sc=False),
)
def _msg_kernel(g_hbm, src_hbm, dst_hbm, zeros_hbm, out_hbm,
                src_v0, src_v1, src_v2, dst_v0, dst_v1, dst_v2, srcx_v, dstx_v,
                msg_v0, msg_v1, msg_v2, msgx_v, acc_sh,
                sem_s0, sem_s1, sem_s2, sem_d0, sem_d1, sem_d2,
                sem_g0, sem_g1, sem_g2, sem_a0, sem_a1, sem_a2):
    cid = lax.axis_index("c")
    sid = lax.axis_index("s")
    wid = cid * NS + sid

    src_v = (src_v0, src_v1, src_v2)
    dst_v = (dst_v0, dst_v1, dst_v2)
    msg_v = (msg_v0, msg_v1, msg_v2)
    sem_s = (sem_s0, sem_s1, sem_s2)
    sem_d = (sem_d0, sem_d1, sem_d2)
    sem_g = (sem_g0, sem_g1, sem_g2)
    sem_a = (sem_a0, sem_a1, sem_a2)

    row_base = jnp.where(wid < 8, wid * 157, 1256 + (wid - 8) * 156)

    def start_idx(s, k):
        r = row_base + 2 * s
        pltpu.async_copy(src_hbm.at[pl.ds(r, 2)], src_v[k], sem_s[k])
        pltpu.async_copy(dst_hbm.at[pl.ds(r, 2)], dst_v[k], sem_d[k])

    def wait_idx(k):
        pltpu.make_async_copy(src_hbm.at[pl.ds(0, 2)], src_v[k], sem_s[k]).wait()
        pltpu.make_async_copy(dst_hbm.at[pl.ds(0, 2)], dst_v[k], sem_d[k]).wait()

    def start_gather(k):
        for j in (0, 1):
            pltpu.async_copy(g_hbm.at[src_v[k].at[j]],
                             msg_v[k].at[pl.ds(j * CH, CH)], sem_g[k])

    def wait_gather(k):
        for j in (0, 1):
            pltpu.make_async_copy(g_hbm.at[src_v[k].at[j]],
                                  msg_v[k].at[pl.ds(j * CH, CH)], sem_g[k]).wait()

    def start_scatter(k):
        for j in (0, 1):
            pltpu.async_copy(msg_v[k].at[pl.ds(j * CH, CH)],
                             acc_sh.at[dst_v[k].at[j]], sem_a[k], add=True)

    def wait_scatter(k):
        for j in (0, 1):
            pltpu.make_async_copy(msg_v[k].at[pl.ds(j * CH, CH)],
                                  acc_sh.at[dst_v[k].at[j]], sem_a[k]).wait()

    @pl.when(sid < NS - 1)
    def _():
        pltpu.sync_copy(zeros_hbm.at[pl.ds(sid * RPT, RPT)],
                        acc_sh.at[pl.ds(sid * RPT, RPT)])

    @pl.when(sid == NS - 1)
    def _():
        pltpu.sync_copy(zeros_hbm.at[pl.ds((NS - 1) * RPT, RPT_LAST)],
                        acc_sh.at[pl.ds((NS - 1) * RPT, RPT_LAST)])

    plsc.subcore_barrier()

    start_idx(0, 0)
    start_idx(1, 1)
    wait_idx(0)
    start_gather(0)

    def outer(m, _):
        for b in (0, 1, 2):
            s = 3 * m + b
            b1 = (b + 1) % 3
            b2 = (b + 2) % 3

            wait_gather(b)
            start_scatter(b)

            @pl.when(s < SLOTS - 1)
            def _():
                wait_idx(b1)
                start_gather(b1)

            @pl.when(s >= 1)
            def _():
                wait_scatter(b2)

            @pl.when(s < SLOTS - 2)
            def _():
                start_idx(s + 2, b2)
        return 0

    lax.fori_loop(0, SLOTS // 3, outer, 0)

    wait_scatter((SLOTS - 1) % 3)

    @pl.when(wid < 8)
    def _():
        r = row_base + 2 * SLOTS
        pltpu.sync_copy(src_hbm.at[pl.ds(r, 1)], srcx_v)
        pltpu.sync_copy(dst_hbm.at[pl.ds(r, 1)], dstx_v)
        pltpu.async_copy(g_hbm.at[srcx_v.at[0]], msgx_v, sem_g0).wait()
        pltpu.sync_copy(msgx_v, acc_sh.at[dstx_v.at[0]], add=True)

    plsc.subcore_barrier()

    @pl.when(sid < NS - 1)
    def _():
        pltpu.sync_copy(acc_sh.at[pl.ds(sid * RPT, RPT)],
                        out_hbm.at[cid, pl.ds(sid * RPT, RPT)])

    @pl.when(sid == NS - 1)
    def _():
        pltpu.sync_copy(acc_sh.at[pl.ds((NS - 1) * RPT, RPT_LAST)],
                        out_hbm.at[cid, pl.ds((NS - 1) * RPT, RPT_LAST)])


def _dinv_from(degt_ref):
    deg = jnp.sum(degt_ref[...], axis=1, keepdims=True) + 1.0
    return lax.rsqrt(deg)


def _tc_pre_body(x_ref, w_ref, degt_ref, g_ref):
    dinv = _dinv_from(degt_ref)
    h = jnp.dot(x_ref[...], w_ref[...], preferred_element_type=jnp.float32)
    g_ref[...] = h * dinv


_tc_pre = pl.pallas_call(
    _tc_pre_body,
    out_shape=jax.ShapeDtypeStruct((N, HIDDEN), jnp.float32),
)


def _ln_relu(u, lw, lb):
    m = jnp.mean(u)
    xc = u - m
    v = jnp.mean(xc * xc)
    yn = xc / (jnp.sqrt(v) + EPS) * lw + lb
    return jnp.maximum(yn, 0.0)


def _tc_mid_body(s_ref, g_ref, degt_ref, b_ref, lw_ref, lb_ref, w2_ref, out_ref):
    dinv = _dinv_from(degt_ref)
    u = (s_ref[0] + s_ref[1] + g_ref[...]) * dinv + b_ref[...]
    yr = _ln_relu(u, lw_ref[...], lb_ref[...])
    h2 = jnp.dot(yr, w2_ref[...], preferred_element_type=jnp.float32)
    out_ref[...] = h2 * dinv


_tc_mid = pl.pallas_call(
    _tc_mid_body,
    out_shape=jax.ShapeDtypeStruct((N, HIDDEN), jnp.float32),
)


def _tc_fin_body(s_ref, g_ref, degt_ref, b_ref, lw_ref, lb_ref, wh_ref, bh_ref,
                 out_ref):
    dinv = _dinv_from(degt_ref)
    u = (s_ref[0] + s_ref[1] + g_ref[...]) * dinv + b_ref[...]
    yr = _ln_relu(u, lw_ref[...], lb_ref[...])
    out_ref[...] = jnp.dot(yr, wh_ref[...], preferred_element_type=jnp.float32) + bh_ref[...]


_tc_fin = pl.pallas_call(
    _tc_fin_body,
    out_shape=jax.ShapeDtypeStruct((N, 1), jnp.float32),
)


def kernel(x, edge_index, batch, W1, b1, ln1_w, ln1_b, W2, b2, ln2_w, ln2_b,
           Wh, bh):
    src = edge_index[0]
    dst = edge_index[1]
    src2d = src.reshape(EROWS, CH)
    dst2d = dst.reshape(EROWS, CH)
    zeros64 = jnp.zeros((N, HIDDEN), jnp.float32)

    degp = _deg_kernel(dst)
    degt = degp.T

    b1r = b1.reshape(1, HIDDEN)
    lw1r = ln1_w.reshape(1, HIDDEN)
    lb1r = ln1_b.reshape(1, HIDDEN)
    b2r = b2.reshape(1, HIDDEN)
    lw2r = ln2_w.reshape(1, HIDDEN)
    lb2r = ln2_b.reshape(1, HIDDEN)
    bhr = bh.reshape(1, 1)

    g1 = _tc_pre(x, W1, degt)
    s1 = _msg_kernel(g1, src2d, dst2d, zeros64)
    g2 = _tc_mid(s1, g1, degt, b1r, lw1r, lb1r, W2)
    s2 = _msg_kernel(g2, src2d, dst2d, zeros64)
    return _tc_fin(s2, g2, degt, b2r, lw2r, lb2r, Wh, bhr)


# speedup vs baseline: 55.7241x; 1.0156x over previous
import functools

import jax
import jax.numpy as jnp
from jax import lax
from jax.experimental import pallas as pl
from jax.experimental.pallas import tpu as pltpu
from jax.experimental.pallas import tpu_sc as plsc

N = 10000
E = 640000
IN_DIM = 128
HIDDEN = 64
EPS = 1e-5

NC = 2
NS = 16
NW = NC * NS
EPW = E // NW
CH = 128
EROWS = E // CH
SLOTS = 78
RPT = 632
RPT_LAST = N - (NS - 1) * RPT

_mesh = plsc.VectorSubcoreMesh(core_axis_name="c", subcore_axis_name="s")


@functools.partial(
    pl.kernel,
    out_type=jax.ShapeDtypeStruct((NW, N), jnp.float32),
    mesh=_mesh,
    scratch_types=[
        pltpu.VMEM((EPW,), jnp.int32),
        pltpu.VMEM((N,), jnp.float32),
        pltpu.SemaphoreType.DMA,
    ],
    compiler_params=pltpu.CompilerParams(needs_layout_passes=False,
                                         use_tc_tiling_on_sc=False),
)
def _deg_kernel(ei_hbm, out_hbm, idx_v, hist_v, sem):
    cid = lax.axis_index("c")
    sid = lax.axis_index("s")
    wid = cid * NS + sid

    pltpu.async_copy(ei_hbm.at[1, pl.ds(wid * EPW, EPW)], idx_v, sem)

    zero16 = jnp.zeros((16,), jnp.float32)

    def zbody(i, _):
        hist_v[pl.ds(i * 16, 16)] = zero16
        return 0

    lax.fori_loop(0, N // 16, zbody, 0)

    pltpu.make_async_copy(ei_hbm.at[1, pl.ds(0, EPW)], idx_v, sem).wait()

    ones16 = jnp.ones((16,), jnp.float32)

    def body(i, _):
        for u in range(5):
            idx16 = idx_v[pl.ds((i * 5 + u) * 16, 16)]
            plsc.addupdate_scatter(hist_v, [idx16], ones16)
        return 0

    lax.fori_loop(0, EPW // (16 * 5), body, 0)

    pltpu.sync_copy(hist_v, out_hbm.at[wid])


@functools.partial(
    pl.kernel,
    out_type=jax.ShapeDtypeStruct((NC, N, HIDDEN), jnp.float32),
    mesh=_mesh,
    scratch_types=[
        pltpu.VMEM((2, CH), jnp.int32),
        pltpu.VMEM((2, CH), jnp.int32),
        pltpu.VMEM((2, CH), jnp.int32),
        pltpu.VMEM((2, CH), jnp.int32),
        pltpu.VMEM((2, CH), jnp.int32),
        pltpu.VMEM((2, CH), jnp.int32),
        pltpu.VMEM((1, CH), jnp.int32),
        pltpu.VMEM((1, CH), jnp.int32),
        pltpu.VMEM((2 * CH, HIDDEN), jnp.float32),
        pltpu.VMEM((2 * CH, HIDDEN), jnp.float32),
        pltpu.VMEM((2 * CH, HIDDEN), jnp.float32),
        pltpu.VMEM((CH, HIDDEN), jnp.float32),
        pltpu.VMEM((2 * CH, HIDDEN), jnp.float32),
        pltpu.VMEM_SHARED((N, HIDDEN), jnp.float32),
        pltpu.SemaphoreType.DMA,
        pltpu.SemaphoreType.DMA,
        pltpu.SemaphoreType.DMA,
        pltpu.SemaphoreType.DMA,
        pltpu.SemaphoreType.DMA,
        pltpu.SemaphoreType.DMA,
        pltpu.SemaphoreType.DMA,
        pltpu.SemaphoreType.DMA,
        pltpu.SemaphoreType.DMA,
        pltpu.SemaphoreType.DMA,
        pltpu.SemaphoreType.DMA,
        pltpu.SemaphoreType.DMA,
    ],
    compiler_params=pltpu.CompilerParams(use_tc_tiling_on_sc=False),
)
def _msg_kernel(g_hbm, ei_hbm, out_hbm,
                src_v0, src_v1, src_v2, dst_v0, dst_v1, dst_v2, srcx_v, dstx_v,
                msg_v0, msg_v1, msg_v2, msgx_v, zbuf_v, acc_sh,
                sem_s0, sem_s1, sem_s2, sem_d0, sem_d1, sem_d2,
                sem_g0, sem_g1, sem_g2, sem_a0, sem_a1, sem_a2):
    cid = lax.axis_index("c")
    sid = lax.axis_index("s")
    wid = cid * NS + sid

    src_v = (src_v0, src_v1, src_v2)
    dst_v = (dst_v0, dst_v1, dst_v2)
    msg_v = (msg_v0, msg_v1, msg_v2)
    sem_s = (sem_s0, sem_s1, sem_s2)
    sem_d = (sem_d0, sem_d1, sem_d2)
    sem_g = (sem_g0, sem_g1, sem_g2)
    sem_a = (sem_a0, sem_a1, sem_a2)

    row_base = jnp.where(wid < 8, wid * 157, 1256 + (wid - 8) * 156)

    def start_idx(s, k):
        e0 = pl.multiple_of((row_base + 2 * s) * CH, 8)
        for j in (0, 1):
            pltpu.async_copy(ei_hbm.at[0, pl.ds(e0 + j * CH, CH)],
                             src_v[k].at[j], sem_s[k])
            pltpu.async_copy(ei_hbm.at[1, pl.ds(e0 + j * CH, CH)],
                             dst_v[k].at[j], sem_d[k])

    def wait_idx(k):
        for j in (0, 1):
            pltpu.make_async_copy(ei_hbm.at[0, pl.ds(0, CH)],
                                  src_v[k].at[j], sem_s[k]).wait()
            pltpu.make_async_copy(ei_hbm.at[1, pl.ds(0, CH)],
                                  dst_v[k].at[j], sem_d[k]).wait()

    def start_gather(k):
        for j in (0, 1):
            pltpu.async_copy(g_hbm.at[src_v[k].at[j]],
                             msg_v[k].at[pl.ds(j * CH, CH)], sem_g[k])

    def wait_gather(k):
        for j in (0, 1):
            pltpu.make_async_copy(g_hbm.at[src_v[k].at[j]],
                                  msg_v[k].at[pl.ds(j * CH, CH)], sem_g[k]).wait()

    def start_scatter(k):
        for j in (0, 1):
            pltpu.async_copy(msg_v[k].at[pl.ds(j * CH, CH)],
                             acc_sh.at[dst_v[k].at[j]], sem_a[k], add=True)

    def wait_scatter(k):
        for j in (0, 1):
            pltpu.make_async_copy(msg_v[k].at[pl.ds(j * CH, CH)],
                                  acc_sh.at[dst_v[k].at[j]], sem_a[k]).wait()

    z16 = jnp.zeros((16,), jnp.float32)

    def zbody(i, _):
        for u in range(4):
            zbuf_v[i, pl.ds(u * 16, 16)] = z16
        return 0

    lax.fori_loop(0, 2 * CH, zbody, 0)

    @pl.when(sid < NS - 1)
    def _():
        for off, sz in ((0, 256), (256, 256), (512, RPT - 512)):
            pltpu.sync_copy(zbuf_v.at[pl.ds(0, sz)],
                            acc_sh.at[pl.ds(sid * RPT + off, sz)])

    @pl.when(sid == NS - 1)
    def _():
        for off, sz in ((0, 256), (256, 256), (512, RPT_LAST - 512)):
            pltpu.sync_copy(zbuf_v.at[pl.ds(0, sz)],
                            acc_sh.at[pl.ds((NS - 1) * RPT + off, sz)])

    plsc.subcore_barrier()

    start_idx(0, 0)
    start_idx(1, 1)
    wait_idx(0)
    start_gather(0)

    def outer(m, _):
        for b in (0, 1, 2):
            s = 3 * m + b
            b1 = (b + 1) % 3
            b2 = (b + 2) % 3

            wait_gather(b)
            start_scatter(b)

            @pl.when(s < SLOTS - 1)
            def _():
                wait_idx(b1)
                start_gather(b1)

            @pl.when(s >= 1)
            def _():
                wait_scatter(b2)

            @pl.when(s < SLOTS - 2)
            def _():
                start_idx(s + 2, b2)
        return 0

    lax.fori_loop(0, SLOTS // 3, outer, 0)

    wait_scatter((SLOTS - 1) % 3)

    @pl.when(wid < 8)
    def _():
        e0 = pl.multiple_of((row_base + 2 * SLOTS) * CH, 8)
        pltpu.sync_copy(ei_hbm.at[0, pl.ds(e0, CH)], srcx_v.at[0])
        pltpu.sync_copy(ei_hbm.at[1, pl.ds(e0, CH)], dstx_v.at[0])
        pltpu.async_copy(g_hbm.at[srcx_v.at[0]], msgx_v, sem_g0).wait()
        pltpu.sync_copy(msgx_v, acc_sh.at[dstx_v.at[0]], add=True)

    plsc.subcore_barrier()

    @pl.when(sid < NS - 1)
    def _():
        pltpu.sync_copy(acc_sh.at[pl.ds(sid * RPT, RPT)],
                        out_hbm.at[cid, pl.ds(sid * RPT, RPT)])

    @pl.when(sid == NS - 1)
    def _():
        pltpu.sync_copy(acc_sh.at[pl.ds((NS - 1) * RPT, RPT_LAST)],
                        out_hbm.at[cid, pl.ds((NS - 1) * RPT, RPT_LAST)])


def _dinv_from(degt_ref):
    deg = jnp.sum(degt_ref[...], axis=1, keepdims=True) + 1.0
    return lax.rsqrt(deg)


def _tc_pre_body(x_ref, w_ref, degt_ref, g_ref):
    dinv = _dinv_from(degt_ref)
    h = jnp.dot(x_ref[...], w_ref[...], preferred_element_type=jnp.float32)
    g_ref[...] = h * dinv


_tc_pre = pl.pallas_call(
    _tc_pre_body,
    out_shape=jax.ShapeDtypeStruct((N, HIDDEN), jnp.float32),
)


def _ln_relu(u, lw, lb):
    m = jnp.mean(u)
    xc = u - m
    v = jnp.mean(xc * xc)
    yn = xc / (jnp.sqrt(v) + EPS) * lw + lb
    return jnp.maximum(yn, 0.0)


def _tc_mid_body(s_ref, g_ref, degt_ref, b_ref, lw_ref, lb_ref, w2_ref, out_ref):
    dinv = _dinv_from(degt_ref)
    u = (s_ref[0] + s_ref[1] + g_ref[...]) * dinv + b_ref[...]
    yr = _ln_relu(u, lw_ref[...], lb_ref[...])
    h2 = jnp.dot(yr, w2_ref[...], preferred_element_type=jnp.float32)
    out_ref[...] = h2 * dinv


_tc_mid = pl.pallas_call(
    _tc_mid_body,
    out_shape=jax.ShapeDtypeStruct((N, HIDDEN), jnp.float32),
)


def _tc_fin_body(s_ref, g_ref, degt_ref, b_ref, lw_ref, lb_ref, wh_ref, bh_ref,
                 out_ref):
    dinv = _dinv_from(degt_ref)
    u = (s_ref[0] + s_ref[1] + g_ref[...]) * dinv + b_ref[...]
    yr = _ln_relu(u, lw_ref[...], lb_ref[...])
    out_ref[...] = jnp.dot(yr, wh_ref[...], preferred_element_type=jnp.float32) + bh_ref[...]


_tc_fin = pl.pallas_call(
    _tc_fin_body,
    out_shape=jax.ShapeDtypeStruct((N, 1), jnp.float32),
)


def kernel(x, edge_index, batch, W1, b1, ln1_w, ln1_b, W2, b2, ln2_w, ln2_b,
           Wh, bh):
    degp = _deg_kernel(edge_index)
    degt = degp.T

    b1r = b1.reshape(1, HIDDEN)
    lw1r = ln1_w.reshape(1, HIDDEN)
    lb1r = ln1_b.reshape(1, HIDDEN)
    b2r = b2.reshape(1, HIDDEN)
    lw2r = ln2_w.reshape(1, HIDDEN)
    lb2r = ln2_b.reshape(1, HIDDEN)
    bhr = bh.reshape(1, 1)

    g1 = _tc_pre(x, W1, degt)
    s1 = _msg_kernel(g1, edge_index)
    g2 = _tc_mid(s1, g1, degt, b1r, lw1r, lb1r, W2)
    s2 = _msg_kernel(g2, edge_index)
    return _tc_fin(s2, g2, degt, b2r, lw2r, lb2r, Wh, bhr)


# speedup vs baseline: 59.1005x; 1.0606x over previous
import functools

import jax
import jax.numpy as jnp
from jax import lax
from jax.experimental import pallas as pl
from jax.experimental.pallas import tpu as pltpu
from jax.experimental.pallas import tpu_sc as plsc

N = 10000
E = 640000
IN_DIM = 128
HIDDEN = 64
EPS = 1e-5

NC = 2
NS = 16
NW = NC * NS
EPW = E // NW
CH = 128
EROWS = E // CH
SLOTS = 78
RPT = 632
RPT_LAST = N - (NS - 1) * RPT

_mesh = plsc.VectorSubcoreMesh(core_axis_name="c", subcore_axis_name="s")


@functools.partial(
    pl.kernel,
    out_type=jax.ShapeDtypeStruct((NW, N), jnp.float32),
    mesh=_mesh,
    scratch_types=[
        pltpu.VMEM((EPW,), jnp.int32),
        pltpu.VMEM((N,), jnp.float32),
        pltpu.SemaphoreType.DMA,
    ],
    compiler_params=pltpu.CompilerParams(needs_layout_passes=False,
                                         use_tc_tiling_on_sc=False),
)
def _deg_kernel(ei_hbm, out_hbm, idx_v, hist_v, sem):
    cid = lax.axis_index("c")
    sid = lax.axis_index("s")
    wid = cid * NS + sid

    pltpu.async_copy(ei_hbm.at[1, pl.ds(wid * EPW, EPW)], idx_v, sem)

    zero16 = jnp.zeros((16,), jnp.float32)

    def zbody(i, _):
        hist_v[pl.ds(i * 16, 16)] = zero16
        return 0

    lax.fori_loop(0, N // 16, zbody, 0)

    pltpu.make_async_copy(ei_hbm.at[1, pl.ds(0, EPW)], idx_v, sem).wait()

    ones16 = jnp.ones((16,), jnp.float32)

    def body(i, _):
        for u in range(5):
            idx16 = idx_v[pl.ds((i * 5 + u) * 16, 16)]
            plsc.addupdate_scatter(hist_v, [idx16], ones16)
        return 0

    lax.fori_loop(0, EPW // (16 * 5), body, 0)

    pltpu.sync_copy(hist_v, out_hbm.at[wid])


@functools.partial(
    pl.kernel,
    out_type=jax.ShapeDtypeStruct((N, NC * HIDDEN), jnp.float32),
    mesh=_mesh,
    scratch_types=[
        pltpu.VMEM((2, CH), jnp.int32),
        pltpu.VMEM((2, CH), jnp.int32),
        pltpu.VMEM((2, CH), jnp.int32),
        pltpu.VMEM((2, CH), jnp.int32),
        pltpu.VMEM((2, CH), jnp.int32),
        pltpu.VMEM((2, CH), jnp.int32),
        pltpu.VMEM((1, CH), jnp.int32),
        pltpu.VMEM((1, CH), jnp.int32),
        pltpu.VMEM((2 * CH, HIDDEN), jnp.float32),
        pltpu.VMEM((2 * CH, HIDDEN), jnp.float32),
        pltpu.VMEM((2 * CH, HIDDEN), jnp.float32),
        pltpu.VMEM((CH, HIDDEN), jnp.float32),
        pltpu.VMEM((2 * CH, HIDDEN), jnp.float32),
        pltpu.VMEM_SHARED((N, HIDDEN), jnp.float32),
        pltpu.SemaphoreType.DMA,
        pltpu.SemaphoreType.DMA,
        pltpu.SemaphoreType.DMA,
        pltpu.SemaphoreType.DMA,
        pltpu.SemaphoreType.DMA,
        pltpu.SemaphoreType.DMA,
        pltpu.SemaphoreType.DMA,
        pltpu.SemaphoreType.DMA,
        pltpu.SemaphoreType.DMA,
        pltpu.SemaphoreType.DMA,
        pltpu.SemaphoreType.DMA,
        pltpu.SemaphoreType.DMA,
    ],
    compiler_params=pltpu.CompilerParams(use_tc_tiling_on_sc=False),
)
def _msg_kernel(g_hbm, ei_hbm, out_hbm,
                src_v0, src_v1, src_v2, dst_v0, dst_v1, dst_v2, srcx_v, dstx_v,
                msg_v0, msg_v1, msg_v2, msgx_v, zbuf_v, acc_sh,
                sem_s0, sem_s1, sem_s2, sem_d0, sem_d1, sem_d2,
                sem_g0, sem_g1, sem_g2, sem_a0, sem_a1, sem_a2):
    cid = lax.axis_index("c")
    sid = lax.axis_index("s")
    wid = cid * NS + sid

    src_v = (src_v0, src_v1, src_v2)
    dst_v = (dst_v0, dst_v1, dst_v2)
    msg_v = (msg_v0, msg_v1, msg_v2)
    sem_s = (sem_s0, sem_s1, sem_s2)
    sem_d = (sem_d0, sem_d1, sem_d2)
    sem_g = (sem_g0, sem_g1, sem_g2)
    sem_a = (sem_a0, sem_a1, sem_a2)

    row_base = jnp.where(wid < 8, wid * 157, 1256 + (wid - 8) * 156)

    def start_idx(s, k):
        e0 = pl.multiple_of((row_base + 2 * s) * CH, 8)
        for j in (0, 1):
            pltpu.async_copy(ei_hbm.at[0, pl.ds(e0 + j * CH, CH)],
                             src_v[k].at[j], sem_s[k])
            pltpu.async_copy(ei_hbm.at[1, pl.ds(e0 + j * CH, CH)],
                             dst_v[k].at[j], sem_d[k])

    def wait_idx(k):
        for j in (0, 1):
            pltpu.make_async_copy(ei_hbm.at[0, pl.ds(0, CH)],
                                  src_v[k].at[j], sem_s[k]).wait()
            pltpu.make_async_copy(ei_hbm.at[1, pl.ds(0, CH)],
                                  dst_v[k].at[j], sem_d[k]).wait()

    def start_gather(k):
        for j in (0, 1):
            pltpu.async_copy(g_hbm.at[src_v[k].at[j]],
                             msg_v[k].at[pl.ds(j * CH, CH)], sem_g[k])

    def wait_gather(k):
        for j in (0, 1):
            pltpu.make_async_copy(g_hbm.at[src_v[k].at[j]],
                                  msg_v[k].at[pl.ds(j * CH, CH)], sem_g[k]).wait()

    def start_scatter(k):
        for j in (0, 1):
            pltpu.async_copy(msg_v[k].at[pl.ds(j * CH, CH)],
                             acc_sh.at[dst_v[k].at[j]], sem_a[k], add=True)

    def wait_scatter(k):
        for j in (0, 1):
            pltpu.make_async_copy(msg_v[k].at[pl.ds(j * CH, CH)],
                                  acc_sh.at[dst_v[k].at[j]], sem_a[k]).wait()

    z16 = jnp.zeros((16,), jnp.float32)

    def zbody(i, _):
        for u in range(4):
            zbuf_v[i, pl.ds(u * 16, 16)] = z16
        return 0

    lax.fori_loop(0, 2 * CH, zbody, 0)

    @pl.when(sid < NS - 1)
    def _():
        for off, sz in ((0, 256), (256, 256), (512, RPT - 512)):
            pltpu.sync_copy(zbuf_v.at[pl.ds(0, sz)],
                            acc_sh.at[pl.ds(sid * RPT + off, sz)])

    @pl.when(sid == NS - 1)
    def _():
        for off, sz in ((0, 256), (256, 256), (512, RPT_LAST - 512)):
            pltpu.sync_copy(zbuf_v.at[pl.ds(0, sz)],
                            acc_sh.at[pl.ds((NS - 1) * RPT + off, sz)])

    plsc.subcore_barrier()

    start_idx(0, 0)
    start_idx(1, 1)
    wait_idx(0)
    start_gather(0)

    def outer(m, _):
        for b in (0, 1, 2):
            s = 3 * m + b
            b1 = (b + 1) % 3
            b2 = (b + 2) % 3

            wait_gather(b)
            start_scatter(b)

            @pl.when(s < SLOTS - 1)
            def _():
                wait_idx(b1)
                start_gather(b1)

            @pl.when(s >= 1)
            def _():
                wait_scatter(b2)

            @pl.when(s < SLOTS - 2)
            def _():
                start_idx(s + 2, b2)
        return 0

    lax.fori_loop(0, SLOTS // 3, outer, 0)

    wait_scatter((SLOTS - 1) % 3)

    @pl.when(wid < 8)
    def _():
        e0 = pl.multiple_of((row_base + 2 * SLOTS) * CH, 8)
        pltpu.sync_copy(ei_hbm.at[0, pl.ds(e0, CH)], srcx_v.at[0])
        pltpu.sync_copy(ei_hbm.at[1, pl.ds(e0, CH)], dstx_v.at[0])
        pltpu.async_copy(g_hbm.at[srcx_v.at[0]], msgx_v, sem_g0).wait()
        pltpu.sync_copy(msgx_v, acc_sh.at[dstx_v.at[0]], add=True)

    plsc.subcore_barrier()

    @pl.when(sid < NS - 1)
    def _():
        pltpu.sync_copy(acc_sh.at[pl.ds(sid * RPT, RPT)],
                        out_hbm.at[pl.ds(sid * RPT, RPT),
                                   pl.ds(cid * HIDDEN, HIDDEN)])

    @pl.when(sid == NS - 1)
    def _():
        pltpu.sync_copy(acc_sh.at[pl.ds((NS - 1) * RPT, RPT_LAST)],
                        out_hbm.at[pl.ds((NS - 1) * RPT, RPT_LAST),
                                   pl.ds(cid * HIDDEN, HIDDEN)])


def _dinv_from(degp_ref):
    deg = jnp.sum(degp_ref[...], axis=0, keepdims=True) + 1.0
    return jnp.transpose(lax.rsqrt(deg), (1, 0))


def _tc_pre_body(x_ref, w_ref, degp_ref, g_ref):
    dinv = _dinv_from(degp_ref)
    h = jnp.dot(x_ref[...], w_ref[...], preferred_element_type=jnp.float32)
    g_ref[...] = h * dinv


_tc_pre = pl.pallas_call(
    _tc_pre_body,
    out_shape=jax.ShapeDtypeStruct((N, HIDDEN), jnp.float32),
)


def _ln_relu(u, lw, lb):
    m = jnp.mean(u)
    xc = u - m
    v = jnp.mean(xc * xc)
    yn = xc / (jnp.sqrt(v) + EPS) * lw + lb
    return jnp.maximum(yn, 0.0)


def _tc_mid_body(s_ref, g_ref, degp_ref, b_ref, lw_ref, lb_ref, w2_ref, out_ref):
    dinv = _dinv_from(degp_ref)
    u = (s_ref[:, 0:HIDDEN] + s_ref[:, HIDDEN:2 * HIDDEN] + g_ref[...]) * dinv + b_ref[...]
    yr = _ln_relu(u, lw_ref[...], lb_ref[...])
    h2 = jnp.dot(yr, w2_ref[...], preferred_element_type=jnp.float32)
    out_ref[...] = h2 * dinv


_tc_mid = pl.pallas_call(
    _tc_mid_body,
    out_shape=jax.ShapeDtypeStruct((N, HIDDEN), jnp.float32),
)


def _tc_fin_body(s_ref, g_ref, degp_ref, b_ref, lw_ref, lb_ref, wh_ref, bh_ref,
                 out_ref):
    dinv = _dinv_from(degp_ref)
    u = (s_ref[:, 0:HIDDEN] + s_ref[:, HIDDEN:2 * HIDDEN] + g_ref[...]) * dinv + b_ref[...]
    yr = _ln_relu(u, lw_ref[...], lb_ref[...])
    out_ref[...] = jnp.dot(yr, wh_ref[...], preferred_element_type=jnp.float32) + bh_ref[...]


_tc_fin = pl.pallas_call(
    _tc_fin_body,
    out_shape=jax.ShapeDtypeStruct((N, 1), jnp.float32),
)


def kernel(x, edge_index, batch, W1, b1, ln1_w, ln1_b, W2, b2, ln2_w, ln2_b,
           Wh, bh):
    degp = _deg_kernel(edge_index)

    b1r = b1.reshape(1, HIDDEN)
    lw1r = ln1_w.reshape(1, HIDDEN)
    lb1r = ln1_b.reshape(1, HIDDEN)
    b2r = b2.reshape(1, HIDDEN)
    lw2r = ln2_w.reshape(1, HIDDEN)
    lb2r = ln2_b.reshape(1, HIDDEN)
    bhr = bh.reshape(1, 1)

    g1 = _tc_pre(x, W1, degp)
    s1 = _msg_kernel(g1, edge_index)
    g2 = _tc_mid(s1, g1, degp, b1r, lw1r, lb1r, W2)
    s2 = _msg_kernel(g2, edge_index)
    return _tc_fin(s2, g2, degp, b2r, lw2r, lb2r, Wh, bhr)


# speedup vs baseline: 62.1816x; 1.0521x over previous
import functools

import jax
import jax.numpy as jnp
from jax import lax
from jax.experimental import pallas as pl
from jax.experimental.pallas import tpu as pltpu
from jax.experimental.pallas import tpu_sc as plsc

N = 10000
E = 640000
IN_DIM = 128
HIDDEN = 64
EPS = 1e-5

NC = 2
NS = 16
NW = NC * NS
EPW = E // NW
CH = 128
EROWS = E // CH
SLOTS = 156
RPT = 632
RPT_LAST = N - (NS - 1) * RPT

_mesh = plsc.VectorSubcoreMesh(core_axis_name="c", subcore_axis_name="s")


@functools.partial(
    pl.kernel,
    out_type=jax.ShapeDtypeStruct((NW, N), jnp.float32),
    mesh=_mesh,
    scratch_types=[
        pltpu.VMEM((EPW,), jnp.int32),
        pltpu.VMEM((N,), jnp.float32),
        pltpu.SemaphoreType.DMA,
    ],
    compiler_params=pltpu.CompilerParams(needs_layout_passes=False,
                                         use_tc_tiling_on_sc=False),
)
def _deg_kernel(ei_hbm, out_hbm, idx_v, hist_v, sem):
    cid = lax.axis_index("c")
    sid = lax.axis_index("s")
    wid = cid * NS + sid

    pltpu.async_copy(ei_hbm.at[1, pl.ds(wid * EPW, EPW)], idx_v, sem)

    zero16 = jnp.zeros((16,), jnp.float32)

    def zbody(i, _):
        hist_v[pl.ds(i * 16, 16)] = zero16
        return 0

    lax.fori_loop(0, N // 16, zbody, 0)

    pltpu.make_async_copy(ei_hbm.at[1, pl.ds(0, EPW)], idx_v, sem).wait()

    ones16 = jnp.ones((16,), jnp.float32)

    def body(i, _):
        for u in range(5):
            idx16 = idx_v[pl.ds((i * 5 + u) * 16, 16)]
            plsc.addupdate_scatter(hist_v, [idx16], ones16)
        return 0

    lax.fori_loop(0, EPW // (16 * 5), body, 0)

    pltpu.sync_copy(hist_v, out_hbm.at[wid])


@functools.partial(
    pl.kernel,
    out_type=jax.ShapeDtypeStruct((N, NC * HIDDEN), jnp.float32),
    mesh=_mesh,
    scratch_types=[
        pltpu.VMEM((1, CH), jnp.int32),
        pltpu.VMEM((1, CH), jnp.int32),
        pltpu.VMEM((1, CH), jnp.int32),
        pltpu.VMEM((1, CH), jnp.int32),
        pltpu.VMEM((1, CH), jnp.int32),
        pltpu.VMEM((1, CH), jnp.int32),
        pltpu.VMEM((1, CH), jnp.int32),
        pltpu.VMEM((1, CH), jnp.int32),
        pltpu.VMEM((1, CH), jnp.int32),
        pltpu.VMEM((1, CH), jnp.int32),
        pltpu.VMEM((1, CH), jnp.int32),
        pltpu.VMEM((1, CH), jnp.int32),
        pltpu.VMEM((1, CH), jnp.int32),
        pltpu.VMEM((1, CH), jnp.int32),
        pltpu.VMEM((CH, HIDDEN), jnp.float32),
        pltpu.VMEM((CH, HIDDEN), jnp.float32),
        pltpu.VMEM((CH, HIDDEN), jnp.float32),
        pltpu.VMEM((CH, HIDDEN), jnp.float32),
        pltpu.VMEM((CH, HIDDEN), jnp.float32),
        pltpu.VMEM((CH, HIDDEN), jnp.float32),
        pltpu.VMEM((CH, HIDDEN), jnp.float32),
        pltpu.VMEM_SHARED((N, HIDDEN), jnp.float32),
        pltpu.SemaphoreType.DMA,
        pltpu.SemaphoreType.DMA,
        pltpu.SemaphoreType.DMA,
        pltpu.SemaphoreType.DMA,
        pltpu.SemaphoreType.DMA,
        pltpu.SemaphoreType.DMA,
        pltpu.SemaphoreType.DMA,
        pltpu.SemaphoreType.DMA,
        pltpu.SemaphoreType.DMA,
        pltpu.SemaphoreType.DMA,
        pltpu.SemaphoreType.DMA,
        pltpu.SemaphoreType.DMA,
        pltpu.SemaphoreType.DMA,
        pltpu.SemaphoreType.DMA,
        pltpu.SemaphoreType.DMA,
        pltpu.SemaphoreType.DMA,
        pltpu.SemaphoreType.DMA,
        pltpu.SemaphoreType.DMA,
    ],
    compiler_params=pltpu.CompilerParams(use_tc_tiling_on_sc=False),
)
def _msg_kernel(g_hbm, ei_hbm, out_hbm,
                src_v0, src_v1, src_v2, src_v3, src_v4, src_v5,
                dst_v0, dst_v1, dst_v2, dst_v3, dst_v4, dst_v5,
                srcx_v, dstx_v,
                msg_v0, msg_v1, msg_v2, msg_v3, msg_v4, msg_v5, msgx_v, acc_sh,
                sem_i0, sem_i1, sem_i2, sem_i3, sem_i4, sem_i5,
                sem_g0, sem_g1, sem_g2, sem_g3, sem_g4, sem_g5,
                sem_a0, sem_a1, sem_a2, sem_a3, sem_a4, sem_a5):
    cid = lax.axis_index("c")
    sid = lax.axis_index("s")
    wid = cid * NS + sid

    NB = 6
    src_v = (src_v0, src_v1, src_v2, src_v3, src_v4, src_v5)
    dst_v = (dst_v0, dst_v1, dst_v2, dst_v3, dst_v4, dst_v5)
    msg_v = (msg_v0, msg_v1, msg_v2, msg_v3, msg_v4, msg_v5)
    sem_i = (sem_i0, sem_i1, sem_i2, sem_i3, sem_i4, sem_i5)
    sem_g = (sem_g0, sem_g1, sem_g2, sem_g3, sem_g4, sem_g5)
    sem_a = (sem_a0, sem_a1, sem_a2, sem_a3, sem_a4, sem_a5)

    row_base = jnp.where(wid < 8, wid * 157, 1256 + (wid - 8) * 156)

    def start_idx(s, k):
        e0 = pl.multiple_of((row_base + s) * CH, 8)
        pltpu.async_copy(ei_hbm.at[0, pl.ds(e0, CH)], src_v[k].at[0], sem_i[k])
        pltpu.async_copy(ei_hbm.at[1, pl.ds(e0, CH)], dst_v[k].at[0], sem_i[k])

    def wait_idx(k):
        pltpu.make_async_copy(ei_hbm.at[0, pl.ds(0, CH)],
                              src_v[k].at[0], sem_i[k]).wait()
        pltpu.make_async_copy(ei_hbm.at[1, pl.ds(0, CH)],
                              dst_v[k].at[0], sem_i[k]).wait()

    def start_gather(k):
        pltpu.async_copy(g_hbm.at[src_v[k].at[0]], msg_v[k], sem_g[k])

    def wait_gather(k):
        pltpu.make_async_copy(g_hbm.at[src_v[k].at[0]], msg_v[k],
                              sem_g[k]).wait()

    def start_scatter(k):
        pltpu.async_copy(msg_v[k], acc_sh.at[dst_v[k].at[0]], sem_a[k],
                         add=True)

    def wait_scatter(k):
        pltpu.make_async_copy(msg_v[k], acc_sh.at[dst_v[k].at[0]],
                              sem_a[k]).wait()

    z16 = jnp.zeros((16,), jnp.float32)

    def zbody(i, _):
        for u in range(4):
            msg_v0[i, pl.ds(u * 16, 16)] = z16
        return 0

    lax.fori_loop(0, CH, zbody, 0)

    r0 = sid * RPT

    @pl.when((cid == 0) & (sid < NS - 1))
    def _():
        pltpu.sync_copy(g_hbm.at[pl.ds(r0, RPT)], acc_sh.at[pl.ds(r0, RPT)])

    @pl.when((cid == 0) & (sid == NS - 1))
    def _():
        pltpu.sync_copy(g_hbm.at[pl.ds((NS - 1) * RPT, RPT_LAST)],
                        acc_sh.at[pl.ds((NS - 1) * RPT, RPT_LAST)])

    @pl.when((cid == 1) & (sid < NS - 1))
    def _():
        for off, sz in ((0, 128), (128, 128), (256, 128), (384, 128),
                        (512, RPT - 512)):
            pltpu.sync_copy(msg_v0.at[pl.ds(0, sz)],
                            acc_sh.at[pl.ds(r0 + off, sz)])

    @pl.when((cid == 1) & (sid == NS - 1))
    def _():
        for off, sz in ((0, 128), (128, 128), (256, 128), (384, 128),
                        (512, RPT_LAST - 512)):
            pltpu.sync_copy(msg_v0.at[pl.ds(0, sz)],
                            acc_sh.at[pl.ds((NS - 1) * RPT + off, sz)])

    plsc.subcore_barrier()

    start_idx(0, 0)
    start_idx(1, 1)
    start_idx(2, 2)
    wait_idx(0)
    start_gather(0)
    wait_idx(1)
    start_gather(1)

    def outer(m, _):
        for b in range(NB):
            s = NB * m + b

            @pl.when(s >= 3)
            def _():
                wait_scatter((b + 3) % NB)

            @pl.when(s < SLOTS - 3)
            def _():
                start_idx(s + 3, (b + 3) % NB)

            wait_gather(b)

            @pl.when(s < SLOTS - 2)
            def _():
                wait_idx((b + 2) % NB)
                start_gather((b + 2) % NB)

            start_scatter(b)
        return 0

    lax.fori_loop(0, SLOTS // NB, outer, 0)

    wait_scatter((SLOTS - 3) % NB)
    wait_scatter((SLOTS - 2) % NB)
    wait_scatter((SLOTS - 1) % NB)

    @pl.when(wid < 8)
    def _():
        e0 = pl.multiple_of((row_base + SLOTS) * CH, 8)
        pltpu.sync_copy(ei_hbm.at[0, pl.ds(e0, CH)], srcx_v.at[0])
        pltpu.sync_copy(ei_hbm.at[1, pl.ds(e0, CH)], dstx_v.at[0])
        pltpu.async_copy(g_hbm.at[srcx_v.at[0]], msgx_v, sem_g0).wait()
        pltpu.sync_copy(msgx_v, acc_sh.at[dstx_v.at[0]], add=True)

    plsc.subcore_barrier()

    @pl.when(sid < NS - 1)
    def _():
        pltpu.sync_copy(acc_sh.at[pl.ds(sid * RPT, RPT)],
                        out_hbm.at[pl.ds(sid * RPT, RPT),
                                   pl.ds(cid * HIDDEN, HIDDEN)])

    @pl.when(sid == NS - 1)
    def _():
        pltpu.sync_copy(acc_sh.at[pl.ds((NS - 1) * RPT, RPT_LAST)],
                        out_hbm.at[pl.ds((NS - 1) * RPT, RPT_LAST),
                                   pl.ds(cid * HIDDEN, HIDDEN)])


def _dinv_from(degp_ref):
    deg = jnp.sum(degp_ref[...], axis=0, keepdims=True) + 1.0
    return jnp.transpose(lax.rsqrt(deg), (1, 0))


def _tc_pre_body(x_ref, w_ref, degp_ref, g_ref):
    dinv = _dinv_from(degp_ref)
    h = jnp.dot(x_ref[...], w_ref[...], preferred_element_type=jnp.float32)
    g_ref[...] = h * dinv


_tc_pre = pl.pallas_call(
    _tc_pre_body,
    out_shape=jax.ShapeDtypeStruct((N, HIDDEN), jnp.float32),
)


def _ln_relu(u, lw, lb):
    m = jnp.mean(u)
    v = jnp.mean(u * u) - m * m
    xc = u - m
    yn = xc / (jnp.sqrt(v) + EPS) * lw + lb
    return jnp.maximum(yn, 0.0)


def _tc_mid_body(s_ref, degp_ref, b_ref, lw_ref, lb_ref, w2_ref, out_ref):
    dinv = _dinv_from(degp_ref)
    u = (s_ref[:, 0:HIDDEN] + s_ref[:, HIDDEN:2 * HIDDEN]) * dinv + b_ref[...]
    yr = _ln_relu(u, lw_ref[...], lb_ref[...])
    h2 = jnp.dot(yr, w2_ref[...], preferred_element_type=jnp.float32)
    out_ref[...] = h2 * dinv


_tc_mid = pl.pallas_call(
    _tc_mid_body,
    out_shape=jax.ShapeDtypeStruct((N, HIDDEN), jnp.float32),
)


def _tc_fin_body(s_ref, degp_ref, b_ref, lw_ref, lb_ref, wh_ref, bh_ref,
                 out_ref):
    dinv = _dinv_from(degp_ref)
    u = (s_ref[:, 0:HIDDEN] + s_ref[:, HIDDEN:2 * HIDDEN]) * dinv + b_ref[...]
    yr = _ln_relu(u, lw_ref[...], lb_ref[...])
    out_ref[...] = jnp.dot(yr, wh_ref[...], preferred_element_type=jnp.float32) + bh_ref[...]


_tc_fin = pl.pallas_call(
    _tc_fin_body,
    out_shape=jax.ShapeDtypeStruct((N, 1), jnp.float32),
)


def kernel(x, edge_index, batch, W1, b1, ln1_w, ln1_b, W2, b2, ln2_w, ln2_b,
           Wh, bh):
    degp = _deg_kernel(edge_index)

    b1r = b1.reshape(1, HIDDEN)
    lw1r = ln1_w.reshape(1, HIDDEN)
    lb1r = ln1_b.reshape(1, HIDDEN)
    b2r = b2.reshape(1, HIDDEN)
    lw2r = ln2_w.reshape(1, HIDDEN)
    lb2r = ln2_b.reshape(1, HIDDEN)
    bhr = bh.reshape(1, 1)

    g1 = _tc_pre(x, W1, degp)
    s1 = _msg_kernel(g1, edge_index)
    g2 = _tc_mid(s1, degp, b1r, lw1r, lb1r, W2)
    s2 = _msg_kernel(g2, edge_index)
    return _tc_fin(s2, degp, b2r, lw2r, lb2r, Wh, bhr)


# speedup vs baseline: 62.2975x; 1.0019x over previous
import functools

import jax
import jax.numpy as jnp
from jax import lax
from jax.experimental import pallas as pl
from jax.experimental.pallas import tpu as pltpu
from jax.experimental.pallas import tpu_sc as plsc

N = 10000
E = 640000
IN_DIM = 128
HIDDEN = 64
EPS = 1e-5

NC = 2
NS = 16
NW = NC * NS
EPW = E // NW
CH = 128
EROWS = E // CH
SLOTS = 156
RPT = 632
RPT_LAST = N - (NS - 1) * RPT

_mesh = plsc.VectorSubcoreMesh(core_axis_name="c", subcore_axis_name="s")


@functools.partial(
    pl.kernel,
    out_type=jax.ShapeDtypeStruct((NW, N), jnp.float32),
    mesh=_mesh,
    scratch_types=[
        pltpu.VMEM((EPW,), jnp.int32),
        pltpu.VMEM((N,), jnp.float32),
        pltpu.SemaphoreType.DMA,
    ],
    compiler_params=pltpu.CompilerParams(needs_layout_passes=False,
                                         use_tc_tiling_on_sc=False),
)
def _deg_kernel(ei_hbm, out_hbm, idx_v, hist_v, sem):
    cid = lax.axis_index("c")
    sid = lax.axis_index("s")
    wid = cid * NS + sid

    pltpu.async_copy(ei_hbm.at[1, pl.ds(wid * EPW, EPW)], idx_v, sem)

    zero16 = jnp.zeros((16,), jnp.float32)

    def zbody(i, _):
        hist_v[pl.ds(i * 16, 16)] = zero16
        return 0

    lax.fori_loop(0, N // 16, zbody, 0)

    pltpu.make_async_copy(ei_hbm.at[1, pl.ds(0, EPW)], idx_v, sem).wait()

    ones16 = jnp.ones((16,), jnp.float32)

    def body(i, _):
        for u in range(10):
            idx16 = idx_v[pl.ds((i * 10 + u) * 16, 16)]
            plsc.addupdate_scatter(hist_v, [idx16], ones16)
        return 0

    lax.fori_loop(0, EPW // (16 * 10), body, 0)

    pltpu.sync_copy(hist_v, out_hbm.at[wid])


@functools.partial(
    pl.kernel,
    out_type=jax.ShapeDtypeStruct((N, NC * HIDDEN), jnp.float32),
    mesh=_mesh,
    scratch_types=[
        pltpu.VMEM((1, CH), jnp.int32),
        pltpu.VMEM((1, CH), jnp.int32),
        pltpu.VMEM((1, CH), jnp.int32),
        pltpu.VMEM((1, CH), jnp.int32),
        pltpu.VMEM((1, CH), jnp.int32),
        pltpu.VMEM((1, CH), jnp.int32),
        pltpu.VMEM((1, CH), jnp.int32),
        pltpu.VMEM((1, CH), jnp.int32),
        pltpu.VMEM((1, CH), jnp.int32),
        pltpu.VMEM((1, CH), jnp.int32),
        pltpu.VMEM((1, CH), jnp.int32),
        pltpu.VMEM((1, CH), jnp.int32),
        pltpu.VMEM((1, CH), jnp.int32),
        pltpu.VMEM((1, CH), jnp.int32),
        pltpu.VMEM((CH, HIDDEN), jnp.float32),
        pltpu.VMEM((CH, HIDDEN), jnp.float32),
        pltpu.VMEM((CH, HIDDEN), jnp.float32),
        pltpu.VMEM((CH, HIDDEN), jnp.float32),
        pltpu.VMEM((CH, HIDDEN), jnp.float32),
        pltpu.VMEM((CH, HIDDEN), jnp.float32),
        pltpu.VMEM((CH, HIDDEN), jnp.float32),
        pltpu.VMEM_SHARED((N, HIDDEN), jnp.float32),
        pltpu.SemaphoreType.DMA,
        pltpu.SemaphoreType.DMA,
        pltpu.SemaphoreType.DMA,
        pltpu.SemaphoreType.DMA,
        pltpu.SemaphoreType.DMA,
        pltpu.SemaphoreType.DMA,
        pltpu.SemaphoreType.DMA,
        pltpu.SemaphoreType.DMA,
        pltpu.SemaphoreType.DMA,
        pltpu.SemaphoreType.DMA,
        pltpu.SemaphoreType.DMA,
        pltpu.SemaphoreType.DMA,
        pltpu.SemaphoreType.DMA,
        pltpu.SemaphoreType.DMA,
        pltpu.SemaphoreType.DMA,
        pltpu.SemaphoreType.DMA,
        pltpu.SemaphoreType.DMA,
        pltpu.SemaphoreType.DMA,
    ],
    compiler_params=pltpu.CompilerParams(use_tc_tiling_on_sc=False),
)
def _msg_kernel(g_hbm, ei_hbm, out_hbm,
                src_v0, src_v1, src_v2, src_v3, src_v4, src_v5,
                dst_v0, dst_v1, dst_v2, dst_v3, dst_v4, dst_v5,
                srcx_v, dstx_v,
                msg_v0, msg_v1, msg_v2, msg_v3, msg_v4, msg_v5, msgx_v, acc_sh,
                sem_i0, sem_i1, sem_i2, sem_i3, sem_i4, sem_i5,
                sem_g0, sem_g1, sem_g2, sem_g3, sem_g4, sem_g5,
                sem_a0, sem_a1, sem_a2, sem_a3, sem_a4, sem_a5):
    cid = lax.axis_index("c")
    sid = lax.axis_index("s")
    wid = cid * NS + sid

    NB = 6
    src_v = (src_v0, src_v1, src_v2, src_v3, src_v4, src_v5)
    dst_v = (dst_v0, dst_v1, dst_v2, dst_v3, dst_v4, dst_v5)
    msg_v = (msg_v0, msg_v1, msg_v2, msg_v3, msg_v4, msg_v5)
    sem_i = (sem_i0, sem_i1, sem_i2, sem_i3, sem_i4, sem_i5)
    sem_g = (sem_g0, sem_g1, sem_g2, sem_g3, sem_g4, sem_g5)
    sem_a = (sem_a0, sem_a1, sem_a2, sem_a3, sem_a4, sem_a5)

    row_base = jnp.where(wid < 8, wid * 157, 1256 + (wid - 8) * 156)

    def start_idx(s, k):
        e0 = pl.multiple_of((row_base + s) * CH, 8)
        pltpu.async_copy(ei_hbm.at[0, pl.ds(e0, CH)], src_v[k].at[0], sem_i[k])
        pltpu.async_copy(ei_hbm.at[1, pl.ds(e0, CH)], dst_v[k].at[0], sem_i[k])

    def wait_idx(k):
        pltpu.make_async_copy(ei_hbm.at[0, pl.ds(0, CH)],
                              src_v[k].at[0], sem_i[k]).wait()
        pltpu.make_async_copy(ei_hbm.at[1, pl.ds(0, CH)],
                              dst_v[k].at[0], sem_i[k]).wait()

    def start_gather(k):
        pltpu.async_copy(g_hbm.at[src_v[k].at[0]], msg_v[k], sem_g[k])

    def wait_gather(k):
        pltpu.make_async_copy(g_hbm.at[src_v[k].at[0]], msg_v[k],
                              sem_g[k]).wait()

    def start_scatter(k):
        pltpu.async_copy(msg_v[k], acc_sh.at[dst_v[k].at[0]], sem_a[k],
                         add=True)

    def wait_scatter(k):
        pltpu.make_async_copy(msg_v[k], acc_sh.at[dst_v[k].at[0]],
                              sem_a[k]).wait()

    z16 = jnp.zeros((16,), jnp.float32)

    def zbody(i, _):
        for u in range(4):
            msg_v0[i, pl.ds(u * 16, 16)] = z16
        return 0

    lax.fori_loop(0, CH, zbody, 0)

    r0 = sid * RPT

    @pl.when((cid == 0) & (sid < NS - 1))
    def _():
        pltpu.sync_copy(g_hbm.at[pl.ds(r0, RPT)], acc_sh.at[pl.ds(r0, RPT)])

    @pl.when((cid == 0) & (sid == NS - 1))
    def _():
        pltpu.sync_copy(g_hbm.at[pl.ds((NS - 1) * RPT, RPT_LAST)],
                        acc_sh.at[pl.ds((NS - 1) * RPT, RPT_LAST)])

    @pl.when((cid == 1) & (sid < NS - 1))
    def _():
        for off, sz in ((0, 128), (128, 128), (256, 128), (384, 128),
                        (512, RPT - 512)):
            pltpu.sync_copy(msg_v0.at[pl.ds(0, sz)],
                            acc_sh.at[pl.ds(r0 + off, sz)])

    @pl.when((cid == 1) & (sid == NS - 1))
    def _():
        for off, sz in ((0, 128), (128, 128), (256, 128), (384, 128),
                        (512, RPT_LAST - 512)):
            pltpu.sync_copy(msg_v0.at[pl.ds(0, sz)],
                            acc_sh.at[pl.ds((NS - 1) * RPT + off, sz)])

    plsc.subcore_barrier()

    start_idx(0, 0)
    start_idx(1, 1)
    start_idx(2, 2)
    wait_idx(0)
    start_gather(0)
    wait_idx(1)
    start_gather(1)

    def outer(m, _):
        for b in range(NB):
            s = NB * m + b

            @pl.when(s >= 3)
            def _():
                wait_scatter((b + 3) % NB)

            @pl.when(s < SLOTS - 3)
            def _():
                start_idx(s + 3, (b + 3) % NB)

            wait_gather(b)

            @pl.when(s < SLOTS - 2)
            def _():
                wait_idx((b + 2) % NB)
                start_gather((b + 2) % NB)

            start_scatter(b)
        return 0

    lax.fori_loop(0, SLOTS // NB, outer, 0)

    wait_scatter((SLOTS - 3) % NB)
    wait_scatter((SLOTS - 2) % NB)
    wait_scatter((SLOTS - 1) % NB)

    @pl.when(wid < 8)
    def _():
        e0 = pl.multiple_of((row_base + SLOTS) * CH, 8)
        pltpu.sync_copy(ei_hbm.at[0, pl.ds(e0, CH)], srcx_v.at[0])
        pltpu.sync_copy(ei_hbm.at[1, pl.ds(e0, CH)], dstx_v.at[0])
        pltpu.async_copy(g_hbm.at[srcx_v.at[0]], msgx_v, sem_g0).wait()
        pltpu.sync_copy(msgx_v, acc_sh.at[dstx_v.at[0]], add=True)

    plsc.subcore_barrier()

    @pl.when(sid < NS - 1)
    def _():
        pltpu.sync_copy(acc_sh.at[pl.ds(sid * RPT, RPT)],
                        out_hbm.at[pl.ds(sid * RPT, RPT),
                                   pl.ds(cid * HIDDEN, HIDDEN)])

    @pl.when(sid == NS - 1)
    def _():
        pltpu.sync_copy(acc_sh.at[pl.ds((NS - 1) * RPT, RPT_LAST)],
                        out_hbm.at[pl.ds((NS - 1) * RPT, RPT_LAST),
                                   pl.ds(cid * HIDDEN, HIDDEN)])


def _dinv_from(degp_ref):
    deg = jnp.sum(degp_ref[...], axis=0, keepdims=True) + 1.0
    return jnp.transpose(lax.rsqrt(deg), (1, 0))


def _tc_mm_body(x_ref, w_ref, h_ref):
    h_ref[...] = jnp.dot(x_ref[...], w_ref[...],
                         preferred_element_type=jnp.float32)


_tc_mm = pl.pallas_call(
    _tc_mm_body,
    out_shape=jax.ShapeDtypeStruct((N, HIDDEN), jnp.float32),
)


def _tc_scale_body(h_ref, degp_ref, g_ref):
    g_ref[...] = h_ref[...] * _dinv_from(degp_ref)


_tc_scale = pl.pallas_call(
    _tc_scale_body,
    out_shape=jax.ShapeDtypeStruct((N, HIDDEN), jnp.float32),
)


def _ln_relu(u, lw, lb):
    m = jnp.mean(u)
    v = jnp.mean(u * u) - m * m
    xc = u - m
    yn = xc / (jnp.sqrt(v) + EPS) * lw + lb
    return jnp.maximum(yn, 0.0)


def _tc_mid_body(s_ref, degp_ref, b_ref, lw_ref, lb_ref, w2_ref, out_ref):
    dinv = _dinv_from(degp_ref)
    u = (s_ref[:, 0:HIDDEN] + s_ref[:, HIDDEN:2 * HIDDEN]) * dinv + b_ref[...]
    yr = _ln_relu(u, lw_ref[...], lb_ref[...])
    h2 = jnp.dot(yr, w2_ref[...], preferred_element_type=jnp.float32)
    out_ref[...] = h2 * dinv


_tc_mid = pl.pallas_call(
    _tc_mid_body,
    out_shape=jax.ShapeDtypeStruct((N, HIDDEN), jnp.float32),
)


def _tc_fin_body(s_ref, degp_ref, b_ref, lw_ref, lb_ref, wh_ref, bh_ref,
                 out_ref):
    dinv = _dinv_from(degp_ref)
    u = (s_ref[:, 0:HIDDEN] + s_ref[:, HIDDEN:2 * HIDDEN]) * dinv + b_ref[...]
    yr = _ln_relu(u, lw_ref[...], lb_ref[...])
    out_ref[...] = jnp.dot(yr, wh_ref[...], preferred_element_type=jnp.float32) + bh_ref[...]


_tc_fin = pl.pallas_call(
    _tc_fin_body,
    out_shape=jax.ShapeDtypeStruct((N, 1), jnp.float32),
)


def kernel(x, edge_index, batch, W1, b1, ln1_w, ln1_b, W2, b2, ln2_w, ln2_b,
           Wh, bh):
    degp = _deg_kernel(edge_index)

    b1r = b1.reshape(1, HIDDEN)
    lw1r = ln1_w.reshape(1, HIDDEN)
    lb1r = ln1_b.reshape(1, HIDDEN)
    b2r = b2.reshape(1, HIDDEN)
    lw2r = ln2_w.reshape(1, HIDDEN)
    lb2r = ln2_b.reshape(1, HIDDEN)
    bhr = bh.reshape(1, 1)

    h1 = _tc_mm(x, W1)
    g1 = _tc_scale(h1, degp)
    s1 = _msg_kernel(g1, edge_index)
    g2 = _tc_mid(s1, degp, b1r, lw1r, lb1r, W2)
    s2 = _msg_kernel(g2, edge_index)
    return _tc_fin(s2, degp, b2r, lw2r, lb2r, Wh, bhr)


# speedup vs baseline: 71.1514x; 1.1421x over previous
import functools

import jax
import jax.numpy as jnp
from jax import lax
from jax.experimental import pallas as pl
from jax.experimental.pallas import tpu as pltpu
from jax.experimental.pallas import tpu_sc as plsc

N = 10000
E = 640000
IN_DIM = 128
HIDDEN = 64
EPS = 1e-5

NC = 2
NS = 16
NW = NC * NS
EPW = E // NW
CH = 128
EROWS = E // CH
SLOTS = 156
RPT = 632
RPT_LAST = N - (NS - 1) * RPT

_mesh = plsc.VectorSubcoreMesh(core_axis_name="c", subcore_axis_name="s")


@functools.partial(
    pl.kernel,
    out_type=jax.ShapeDtypeStruct((NW, N), jnp.float32),
    mesh=_mesh,
    scratch_types=[
        pltpu.VMEM((EPW,), jnp.int32),
        pltpu.VMEM((N,), jnp.float32),
        pltpu.SemaphoreType.DMA,
    ],
    compiler_params=pltpu.CompilerParams(needs_layout_passes=False,
                                         use_tc_tiling_on_sc=False),
)
def _deg_kernel(ei_hbm, out_hbm, idx_v, hist_v, sem):
    cid = lax.axis_index("c")
    sid = lax.axis_index("s")
    wid = cid * NS + sid

    pltpu.async_copy(ei_hbm.at[1, pl.ds(wid * EPW, EPW)], idx_v, sem)

    zero16 = jnp.zeros((16,), jnp.float32)

    def zbody(i, _):
        hist_v[pl.ds(i * 16, 16)] = zero16
        return 0

    lax.fori_loop(0, N // 16, zbody, 0)

    pltpu.make_async_copy(ei_hbm.at[1, pl.ds(0, EPW)], idx_v, sem).wait()

    ones16 = jnp.ones((16,), jnp.float32)

    def body(i, _):
        for u in range(10):
            idx16 = idx_v[pl.ds((i * 10 + u) * 16, 16)]
            plsc.addupdate_scatter(hist_v, [idx16], ones16)
        return 0

    lax.fori_loop(0, EPW // (16 * 10), body, 0)

    pltpu.sync_copy(hist_v, out_hbm.at[wid])


@functools.partial(
    pl.kernel,
    out_type=jax.ShapeDtypeStruct((N, NC * HIDDEN), jnp.float32),
    mesh=_mesh,
    scratch_types=[
        pltpu.VMEM((1, CH), jnp.int32),
        pltpu.VMEM((1, CH), jnp.int32),
        pltpu.VMEM((1, CH), jnp.int32),
        pltpu.VMEM((1, CH), jnp.int32),
        pltpu.VMEM((1, CH), jnp.int32),
        pltpu.VMEM((1, CH), jnp.int32),
        pltpu.VMEM((1, CH), jnp.int32),
        pltpu.VMEM((1, CH), jnp.int32),
        pltpu.VMEM((1, CH), jnp.int32),
        pltpu.VMEM((1, CH), jnp.int32),
        pltpu.VMEM((1, CH), jnp.int32),
        pltpu.VMEM((1, CH), jnp.int32),
        pltpu.VMEM((1, CH), jnp.int32),
        pltpu.VMEM((1, CH), jnp.int32),
        pltpu.VMEM((CH, HIDDEN), jnp.float32),
        pltpu.VMEM((CH, HIDDEN), jnp.float32),
        pltpu.VMEM((CH, HIDDEN), jnp.float32),
        pltpu.VMEM((CH, HIDDEN), jnp.float32),
        pltpu.VMEM((CH, HIDDEN), jnp.float32),
        pltpu.VMEM((CH, HIDDEN), jnp.float32),
        pltpu.VMEM((CH, HIDDEN), jnp.float32),
        pltpu.VMEM_SHARED((N, HIDDEN), jnp.float32),
        pltpu.SemaphoreType.DMA,
        pltpu.SemaphoreType.DMA,
        pltpu.SemaphoreType.DMA,
        pltpu.SemaphoreType.DMA,
        pltpu.SemaphoreType.DMA,
        pltpu.SemaphoreType.DMA,
        pltpu.SemaphoreType.DMA,
        pltpu.SemaphoreType.DMA,
        pltpu.SemaphoreType.DMA,
        pltpu.SemaphoreType.DMA,
        pltpu.SemaphoreType.DMA,
        pltpu.SemaphoreType.DMA,
        pltpu.SemaphoreType.DMA,
        pltpu.SemaphoreType.DMA,
        pltpu.SemaphoreType.DMA,
        pltpu.SemaphoreType.DMA,
        pltpu.SemaphoreType.DMA,
        pltpu.SemaphoreType.DMA,
    ],
    compiler_params=pltpu.CompilerParams(use_tc_tiling_on_sc=False),
)
def _msg_kernel(g_hbm, ei_hbm, out_hbm,
                src_v0, src_v1, src_v2, src_v3, src_v4, src_v5,
                dst_v0, dst_v1, dst_v2, dst_v3, dst_v4, dst_v5,
                srcx_v, dstx_v,
                msg_v0, msg_v1, msg_v2, msg_v3, msg_v4, msg_v5, msgx_v, acc_sh,
                sem_i0, sem_i1, sem_i2, sem_i3, sem_i4, sem_i5,
                sem_g0, sem_g1, sem_g2, sem_g3, sem_g4, sem_g5,
                sem_a0, sem_a1, sem_a2, sem_a3, sem_a4, sem_a5):
    cid = lax.axis_index("c")
    sid = lax.axis_index("s")
    wid = cid * NS + sid

    NB = 6
    src_v = (src_v0, src_v1, src_v2, src_v3, src_v4, src_v5)
    dst_v = (dst_v0, dst_v1, dst_v2, dst_v3, dst_v4, dst_v5)
    msg_v = (msg_v0, msg_v1, msg_v2, msg_v3, msg_v4, msg_v5)
    sem_i = (sem_i0, sem_i1, sem_i2, sem_i3, sem_i4, sem_i5)
    sem_g = (sem_g0, sem_g1, sem_g2, sem_g3, sem_g4, sem_g5)
    sem_a = (sem_a0, sem_a1, sem_a2, sem_a3, sem_a4, sem_a5)

    row_base = jnp.where(wid < 8, wid * 157, 1256 + (wid - 8) * 156)

    def start_idx(s, k):
        e0 = pl.multiple_of((row_base + s) * CH, 8)
        pltpu.async_copy(ei_hbm.at[0, pl.ds(e0, CH)], src_v[k].at[0], sem_i[k])
        pltpu.async_copy(ei_hbm.at[1, pl.ds(e0, CH)], dst_v[k].at[0], sem_i[k])

    def wait_idx(k):
        pltpu.make_async_copy(ei_hbm.at[0, pl.ds(0, CH)],
                              src_v[k].at[0], sem_i[k]).wait()
        pltpu.make_async_copy(ei_hbm.at[1, pl.ds(0, CH)],
                              dst_v[k].at[0], sem_i[k]).wait()

    def start_gather(k):
        pltpu.async_copy(g_hbm.at[src_v[k].at[0]], msg_v[k], sem_g[k])

    def wait_gather(k):
        pltpu.make_async_copy(g_hbm.at[src_v[k].at[0]], msg_v[k],
                              sem_g[k]).wait()

    def start_scatter(k):
        pltpu.async_copy(msg_v[k], acc_sh.at[dst_v[k].at[0]], sem_a[k],
                         add=True)

    def wait_scatter(k):
        pltpu.make_async_copy(msg_v[k], acc_sh.at[dst_v[k].at[0]],
                              sem_a[k]).wait()

    z16 = jnp.zeros((16,), jnp.float32)

    def zbody(i, _):
        for u in range(4):
            msg_v0[i, pl.ds(u * 16, 16)] = z16
        return 0

    lax.fori_loop(0, CH, zbody, 0)

    r0 = sid * RPT

    @pl.when((cid == 0) & (sid < NS - 1))
    def _():
        pltpu.sync_copy(g_hbm.at[pl.ds(r0, RPT)], acc_sh.at[pl.ds(r0, RPT)])

    @pl.when((cid == 0) & (sid == NS - 1))
    def _():
        pltpu.sync_copy(g_hbm.at[pl.ds((NS - 1) * RPT, RPT_LAST)],
                        acc_sh.at[pl.ds((NS - 1) * RPT, RPT_LAST)])

    @pl.when((cid == 1) & (sid < NS - 1))
    def _():
        for off, sz in ((0, 128), (128, 128), (256, 128), (384, 128),
                        (512, RPT - 512)):
            pltpu.sync_copy(msg_v0.at[pl.ds(0, sz)],
                            acc_sh.at[pl.ds(r0 + off, sz)])

    @pl.when((cid == 1) & (sid == NS - 1))
    def _():
        for off, sz in ((0, 128), (128, 128), (256, 128), (384, 128),
                        (512, RPT_LAST - 512)):
            pltpu.sync_copy(msg_v0.at[pl.ds(0, sz)],
                            acc_sh.at[pl.ds((NS - 1) * RPT + off, sz)])

    plsc.subcore_barrier()

    start_idx(0, 0)
    start_idx(1, 1)
    start_idx(2, 2)
    start_idx(3, 3)
    wait_idx(0)
    start_gather(0)
    wait_idx(1)
    start_gather(1)
    wait_idx(2)
    start_gather(2)

    def outer(m, _):
        for b in range(NB):
            s = NB * m + b

            @pl.when(s >= 2)
            def _():
                wait_scatter((b + 4) % NB)

            @pl.when(s < SLOTS - 4)
            def _():
                start_idx(s + 4, (b + 4) % NB)

            wait_gather(b)

            @pl.when(s < SLOTS - 3)
            def _():
                wait_idx((b + 3) % NB)
                start_gather((b + 3) % NB)

            start_scatter(b)
        return 0

    lax.fori_loop(0, SLOTS // NB, outer, 0)

    wait_scatter((SLOTS - 2) % NB)
    wait_scatter((SLOTS - 1) % NB)

    @pl.when(wid < 8)
    def _():
        e0 = pl.multiple_of((row_base + SLOTS) * CH, 8)
        pltpu.sync_copy(ei_hbm.at[0, pl.ds(e0, CH)], srcx_v.at[0])
        pltpu.sync_copy(ei_hbm.at[1, pl.ds(e0, CH)], dstx_v.at[0])
        pltpu.async_copy(g_hbm.at[srcx_v.at[0]], msgx_v, sem_g0).wait()
        pltpu.sync_copy(msgx_v, acc_sh.at[dstx_v.at[0]], add=True)

    plsc.subcore_barrier()

    @pl.when(sid < NS - 1)
    def _():
        pltpu.sync_copy(acc_sh.at[pl.ds(sid * RPT, RPT)],
                        out_hbm.at[pl.ds(sid * RPT, RPT),
                                   pl.ds(cid * HIDDEN, HIDDEN)])

    @pl.when(sid == NS - 1)
    def _():
        pltpu.sync_copy(acc_sh.at[pl.ds((NS - 1) * RPT, RPT_LAST)],
                        out_hbm.at[pl.ds((NS - 1) * RPT, RPT_LAST),
                                   pl.ds(cid * HIDDEN, HIDDEN)])


def _dinv_from(degp_ref):
    deg = jnp.sum(degp_ref[...], axis=0, keepdims=True) + 1.0
    return jnp.transpose(lax.rsqrt(deg), (1, 0))


def _tc_mm_body(x_ref, w_ref, h_ref):
    h_ref[...] = jnp.dot(x_ref[...], w_ref[...],
                         preferred_element_type=jnp.float32)


_tc_mm = pl.pallas_call(
    _tc_mm_body,
    out_shape=jax.ShapeDtypeStruct((N, HIDDEN), jnp.float32),
)


def _tc_scale_body(h_ref, degp_ref, g_ref):
    g_ref[...] = h_ref[...] * _dinv_from(degp_ref)


_tc_scale = pl.pallas_call(
    _tc_scale_body,
    out_shape=jax.ShapeDtypeStruct((N, HIDDEN), jnp.float32),
)


def _ln_relu(u, lw, lb):
    m = jnp.mean(u)
    v = jnp.mean(u * u) - m * m
    xc = u - m
    yn = xc / (jnp.sqrt(v) + EPS) * lw + lb
    return jnp.maximum(yn, 0.0)


def _tc_mid_body(s_ref, degp_ref, b_ref, lw_ref, lb_ref, w2_ref, out_ref):
    dinv = _dinv_from(degp_ref)
    u = (s_ref[:, 0:HIDDEN] + s_ref[:, HIDDEN:2 * HIDDEN]) * dinv + b_ref[...]
    yr = _ln_relu(u, lw_ref[...], lb_ref[...])
    h2 = jnp.dot(yr, w2_ref[...], preferred_element_type=jnp.float32)
    out_ref[...] = h2 * dinv


_tc_mid = pl.pallas_call(
    _tc_mid_body,
    out_shape=jax.ShapeDtypeStruct((N, HIDDEN), jnp.float32),
)


def _tc_fin_body(s_ref, degp_ref, b_ref, lw_ref, lb_ref, wh_ref, bh_ref,
                 out_ref):
    dinv = _dinv_from(degp_ref)
    u = (s_ref[:, 0:HIDDEN] + s_ref[:, HIDDEN:2 * HIDDEN]) * dinv + b_ref[...]
    yr = _ln_relu(u, lw_ref[...], lb_ref[...])
    out_ref[...] = jnp.dot(yr, wh_ref[...], preferred_element_type=jnp.float32) + bh_ref[...]


_tc_fin = pl.pallas_call(
    _tc_fin_body,
    out_shape=jax.ShapeDtypeStruct((N, 1), jnp.float32),
)


def kernel(x, edge_index, batch, W1, b1, ln1_w, ln1_b, W2, b2, ln2_w, ln2_b,
           Wh, bh):
    degp = _deg_kernel(edge_index)

    b1r = b1.reshape(1, HIDDEN)
    lw1r = ln1_w.reshape(1, HIDDEN)
    lb1r = ln1_b.reshape(1, HIDDEN)
    b2r = b2.reshape(1, HIDDEN)
    lw2r = ln2_w.reshape(1, HIDDEN)
    lb2r = ln2_b.reshape(1, HIDDEN)
    bhr = bh.reshape(1, 1)

    h1 = _tc_mm(x, W1)
    g1 = _tc_scale(h1, degp)
    s1 = _msg_kernel(g1, edge_index)
    g2 = _tc_mid(s1, degp, b1r, lw1r, lb1r, W2)
    s2 = _msg_kernel(g2, edge_index)
    return _tc_fin(s2, degp, b2r, lw2r, lb2r, Wh, bhr)
